# baseline (speedup 1.0000x reference)
import functools
import jax
import jax.numpy as jnp
from jax.experimental import pallas as pl
from jax.experimental.pallas import tpu as pltpu

N = 10000
E = 320000
R = 1000
D = 128
DEPTH = 2


def _tanh_body(x_ref, o_ref):
    o_ref[...] = jnp.tanh(x_ref[...])


def _tc_tanh(x):
    n = x.shape[0]
    blk = 1000
    return pl.pallas_call(
        _tanh_body,
        grid=(n // blk,),
        in_specs=[pl.BlockSpec((blk, D), lambda i: (i, 0))],
        out_specs=pl.BlockSpec((blk, D), lambda i: (i, 0)),
        out_shape=jax.ShapeDtypeStruct((n, D), jnp.float32),
    )(x)


def kernel(features, rel_emb, adj, r_index, r_val, k0, k1, W_attn, b_attn, W_gate, b_gate):
    src, dst = adj[0], adj[1]
    feats = _tc_tanh(features)
    g = jax.ops.segment_sum(r_val[:, None] * rel_emb[r_index[1]], r_index[0], num_segments=R)
    nrm = jnp.sqrt(jnp.sum(g * g, axis=1, keepdims=True))
    t = g / jnp.maximum(nrm, 1e-12)
    ks = jnp.concatenate([k0, k1], axis=1)
    att_s = t @ ks
    wa = W_attn[0]
    wg = W_gate[0]
    Wp = jnp.stack([wa[:D] + wa[2 * D:], wa[D:2 * D] - wa[2 * D:],
                    wg[:D] + wg[2 * D:], wg[D:2 * D] - wg[2 * D:]], axis=1)
    outc = [feats]
    outs = [feats]
    for l in range(DEPTH):
        av = jnp.concatenate([jnp.exp(att_s[:, l]), jnp.ones((E - R,), jnp.float32)])
        denom1 = jax.ops.segment_sum(av, src, num_segments=N)
        sv = av / denom1[src]
        acc = jax.ops.segment_sum(sv[:, None] * feats[dst], src, num_segments=N)
        dote = jnp.sum(feats[dst[:R]] * t, axis=1)
        corr = (-2.0 * sv[:R] * dote)[:, None] * t
        acc = acc + jax.ops.segment_sum(corr, src[:R], num_segments=N)
        feats = _tc_tanh(acc)
        outc.append(feats)
        P = feats @ Wp
        att = jnp.maximum(jax.nn.sigmoid(P[src, 0] + P[dst, 1] + b_attn[0]), 1e-4)
        gate = jax.nn.sigmoid(P[src, 2] + P[dst, 3] + b_gate[0])
        final = gate * att + (1.0 - gate) * sv
        ef = jnp.exp(final)
        denom2 = jax.ops.segment_sum(ef, src, num_segments=N)
        att2 = ef / denom2[src]
        S1 = jax.ops.segment_sum(att2, src, num_segments=N)
        acc2 = jax.ops.segment_sum(att2[:, None] * feats[dst], src, num_segments=N)
        outs.append(_tc_tanh(feats * S1[:, None] - acc2))
    return (jnp.concatenate(outc, axis=-1), jnp.concatenate(outs, axis=-1))


# speedup vs baseline: 1.0443x; 1.0443x over previous
import functools
import jax
import jax.numpy as jnp
from jax import lax
from jax.experimental import pallas as pl
from jax.experimental.pallas import tpu as pltpu
from jax.experimental.pallas import tpu_sc as plsc

N = 10000
E = 320000
R = 1000
D = 128
DEPTH = 2

NC = 2
NS = 16
L = 16

CK = 128
EPT = 10112
NCHUNK = EPT // CK
E_PAD = EPT * NC * NS
NPS = 624
NTAIL = N - NPS * NS

_mesh = plsc.VectorSubcoreMesh(core_axis_name="c", subcore_axis_name="s",
                               num_cores=NC, num_subcores=NS)


def _agg_body(feats_hbm, wv_hbm, src_hbm, dst_hbm, denom_hbm, zeros_hbm,
              out_hbm, denom_v, src_v, dst_v, av_v, w_v, rows_v, acc_sh, sem):
    c = lax.axis_index("c")
    s = lax.axis_index("s")
    wid = c * NS + s
    pltpu.sync_copy(zeros_hbm.at[pl.ds(s * NPS, NPS)], acc_sh.at[pl.ds(s * NPS, NPS)])

    @pl.when(s == NS - 1)
    def _():
        pltpu.sync_copy(zeros_hbm.at[pl.ds(NPS * NS, NTAIL)],
                        acc_sh.at[pl.ds(NPS * NS, NTAIL)])
    pltpu.sync_copy(denom_hbm, denom_v)
    plsc.subcore_barrier()
    base0 = wid * EPT

    def chunk_body(ci, _):
        base = base0 + ci * CK
        pltpu.sync_copy(src_hbm.at[pl.ds(base, CK)], src_v)
        pltpu.sync_copy(dst_hbm.at[pl.ds(base, CK)], dst_v)
        pltpu.sync_copy(wv_hbm.at[pl.ds(base, CK)], av_v)
        gat = pltpu.async_copy(feats_hbm.at[dst_v], rows_v, sem)

        def wg(gi, _):
            sl = pl.ds(gi * L, L)
            dv = plsc.load_gather(denom_v, [src_v[sl]])
            w_v[sl] = av_v[sl] / dv
            return 0

        lax.fori_loop(0, CK // L, wg, 0)
        gat.wait()

        def rs(j, _):
            wbc = plsc.load_gather(w_v, [jnp.full((L,), j, jnp.int32)])
            for k in range(D // L):
                slk = pl.ds(k * L, L)
                rows_v[j, slk] = rows_v[j, slk] * wbc
            return 0

        lax.fori_loop(0, CK, rs, 0)
        pltpu.sync_copy(rows_v, acc_sh.at[src_v], add=True)
        return 0

    lax.fori_loop(0, NCHUNK, chunk_body, 0)
    plsc.subcore_barrier()
    pltpu.sync_copy(acc_sh.at[pl.ds(s * NPS, NPS)],
                    out_hbm.at[c, pl.ds(s * NPS, NPS)])

    @pl.when(s == NS - 1)
    def _():
        pltpu.sync_copy(acc_sh.at[pl.ds(NPS * NS, NTAIL)],
                        out_hbm.at[c, pl.ds(NPS * NS, NTAIL)])


@functools.partial(
    pl.kernel,
    out_type=jax.ShapeDtypeStruct((NC, N, D), jnp.float32),
    mesh=_mesh,
    compiler_params=pltpu.CompilerParams(needs_layout_passes=False),
    scratch_types=[
        pltpu.VMEM((N,), jnp.float32),
        pltpu.VMEM((CK,), jnp.int32),
        pltpu.VMEM((CK,), jnp.int32),
        pltpu.VMEM((CK,), jnp.float32),
        pltpu.VMEM((CK,), jnp.float32),
        pltpu.VMEM((CK, D), jnp.float32),
        pltpu.VMEM_SHARED((N, D), jnp.float32),
        pltpu.SemaphoreType.DMA,
    ],
)
def _sc_edge_aggregate(*args):
    _agg_body(*args)


def _tanh_body(x_ref, o_ref):
    o_ref[...] = jnp.tanh(x_ref[...])


def _tc_tanh(x):
    n = x.shape[0]
    blk = 1000
    return pl.pallas_call(
        _tanh_body,
        grid=(n // blk,),
        in_specs=[pl.BlockSpec((blk, D), lambda i: (i, 0))],
        out_specs=pl.BlockSpec((blk, D), lambda i: (i, 0)),
        out_shape=jax.ShapeDtypeStruct((n, D), jnp.float32),
    )(x)


def _pad_e(x, fill):
    return jnp.concatenate([x, jnp.full((E_PAD - E,), fill, x.dtype)])


def kernel(features, rel_emb, adj, r_index, r_val, k0, k1, W_attn, b_attn, W_gate, b_gate):
    src, dst = adj[0], adj[1]
    src_p = _pad_e(src, 0)
    dst_p = _pad_e(dst, 0)
    zeros_nd = jnp.zeros((N, D), jnp.float32)
    feats = _tc_tanh(features)
    g = jax.ops.segment_sum(r_val[:, None] * rel_emb[r_index[1]], r_index[0], num_segments=R)
    nrm = jnp.sqrt(jnp.sum(g * g, axis=1, keepdims=True))
    t = g / jnp.maximum(nrm, 1e-12)
    ks = jnp.concatenate([k0, k1], axis=1)
    att_s = t @ ks
    wa = W_attn[0]
    wg = W_gate[0]
    Wp = jnp.stack([wa[:D] + wa[2 * D:], wa[D:2 * D] - wa[2 * D:],
                    wg[:D] + wg[2 * D:], wg[D:2 * D] - wg[2 * D:]], axis=1)
    outc = [feats]
    outs = [feats]
    for l in range(DEPTH):
        av = jnp.concatenate([jnp.exp(att_s[:, l]), jnp.ones((E - R,), jnp.float32)])
        denom1 = jax.ops.segment_sum(av, src, num_segments=N)
        sv = av / denom1[src]
        av_p = _pad_e(av, 0.0)
        parts = _sc_edge_aggregate(feats, av_p, src_p, dst_p, denom1, zeros_nd)
        acc = parts[0] + parts[1]
        dote = jnp.sum(feats[dst[:R]] * t, axis=1)
        corr = (-2.0 * sv[:R] * dote)[:, None] * t
        acc = acc + jax.ops.segment_sum(corr, src[:R], num_segments=N)
        feats = _tc_tanh(acc)
        outc.append(feats)
        P = feats @ Wp
        att = jnp.maximum(jax.nn.sigmoid(P[src, 0] + P[dst, 1] + b_attn[0]), 1e-4)
        gate = jax.nn.sigmoid(P[src, 2] + P[dst, 3] + b_gate[0])
        final = gate * att + (1.0 - gate) * sv
        ef = jnp.exp(final)
        denom2 = jax.ops.segment_sum(ef, src, num_segments=N)
        att2 = ef / denom2[src]
        S1 = jax.ops.segment_sum(att2, src, num_segments=N)
        ef_p = _pad_e(ef, 0.0)
        parts2 = _sc_edge_aggregate(feats, ef_p, src_p, dst_p, denom2, zeros_nd)
        acc2 = parts2[0] + parts2[1]
        outs.append(_tc_tanh(feats * S1[:, None] - acc2))
    return (jnp.concatenate(outc, axis=-1), jnp.concatenate(outs, axis=-1))


# speedup vs baseline: 12.4723x; 11.9438x over previous
import functools
import jax
import jax.numpy as jnp
from jax import lax
from jax.experimental import pallas as pl
from jax.experimental.pallas import tpu as pltpu
from jax.experimental.pallas import tpu_sc as plsc

N = 10000
E = 320000
R = 1000
D = 128
DEPTH = 2

NC = 2
NS = 16
L = 16

CK = 128
EPT = 10112
E_PAD = EPT * NC * NS
EPSC = E_PAD // NS
N_PAD = 10240
NPS = N_PAD // NS
R_PAD = 1024
RPS = R_PAD // NS
CPT = R_PAD // (NC * NS)
PAD_SRC = N_PAD - 1

_mesh = plsc.VectorSubcoreMesh(core_axis_name="c", subcore_axis_name="s",
                               num_cores=NC, num_subcores=NS)
_params = pltpu.CompilerParams(needs_layout_passes=False)


def _sigmoid(x):
    return 1.0 / (1.0 + jnp.exp(-x))


def _scale_rows(rows_v, w_v, nrows):
    def body(j, _):
        wbc = plsc.load_gather(w_v, [jnp.full((L,), j, jnp.int32)])
        for k in range(D // L):
            sl = pl.ds(k * L, L)
            rows_v[j, sl] = rows_v[j, sl] * wbc
        return 0

    lax.fori_loop(0, nrows, body, 0)


def _phase_a_body(rel_hbm, r0_hbm, r1_hbm, rv_hbm, zrows_hbm, out_hbm,
                  i0_v, i1_v, w_v, rows_v, acc_sh, sem):
    c = lax.axis_index("c")
    s = lax.axis_index("s")
    wid = c * NS + s
    pltpu.sync_copy(zrows_hbm.at[pl.ds(s * RPS, RPS)],
                    acc_sh.at[pl.ds(s * RPS, RPS)])
    plsc.subcore_barrier()
    base0 = wid * EPT

    def chunk(ci, _):
        base = base0 + ci * CK
        pltpu.sync_copy(r0_hbm.at[pl.ds(base, CK)], i0_v)
        pltpu.sync_copy(r1_hbm.at[pl.ds(base, CK)], i1_v)
        pltpu.sync_copy(rv_hbm.at[pl.ds(base, CK)], w_v)
        pltpu.async_copy(rel_hbm.at[i1_v], rows_v, sem).wait()
        _scale_rows(rows_v, w_v, CK)
        pltpu.sync_copy(rows_v, acc_sh.at[i0_v], add=True)
        return 0

    lax.fori_loop(0, E_PAD // (NC * NS * CK), chunk, 0)
    plsc.subcore_barrier()
    pltpu.sync_copy(acc_sh.at[pl.ds(s * RPS, RPS)],
                    out_hbm.at[c, pl.ds(s * RPS, RPS)])


_sc_phase_a = functools.partial(
    pl.kernel,
    out_type=jax.ShapeDtypeStruct((NC, R_PAD, D), jnp.float32),
    mesh=_mesh,
    compiler_params=_params,
    scratch_types=[
        pltpu.VMEM((CK,), jnp.int32),
        pltpu.VMEM((CK,), jnp.int32),
        pltpu.VMEM((CK,), jnp.float32),
        pltpu.VMEM((CK, D), jnp.float32),
        pltpu.VMEM_SHARED((R_PAD, D), jnp.float32),
        pltpu.SemaphoreType.DMA,
    ],
)(_phase_a_body)


def _pass_c_body(feats_hbm, av_hbm, src_hbm, dst_hbm, t_hbm,
                 zn_hbm, zrows_hbm, acc_out, den_out,
                 denom_v, src_v, dst_v, av_v, w_v, rows_v,
                 csrc_v, cdst_v, cav_v, cw_v, t_v, crows_v,
                 acc_sh, den_sh, sem):
    c = lax.axis_index("c")
    s = lax.axis_index("s")
    wid = c * NS + s
    pltpu.sync_copy(zrows_hbm.at[pl.ds(s * NPS, NPS)],
                    acc_sh.at[pl.ds(s * NPS, NPS)])
    pltpu.sync_copy(zn_hbm.at[pl.ds(s * NPS, NPS)],
                    den_sh.at[pl.ds(s * NPS, NPS)])
    plsc.subcore_barrier()

    sbase0 = s * EPSC

    def schunk(ci, _):
        base = sbase0 + ci * CK
        pltpu.sync_copy(src_hbm.at[pl.ds(base, CK)], src_v)
        pltpu.sync_copy(av_hbm.at[pl.ds(base, CK)], av_v)
        pltpu.sync_copy(av_v, den_sh.at[src_v], add=True)
        return 0

    lax.fori_loop(0, EPSC // CK, schunk, 0)
    plsc.subcore_barrier()
    pltpu.sync_copy(den_sh, denom_v)
    pltpu.sync_copy(den_sh.at[pl.ds(s * NPS, NPS)],
                    den_out.at[c, pl.ds(s * NPS, NPS)])

    cbase = wid * CPT
    pltpu.sync_copy(src_hbm.at[pl.ds(cbase, CPT)], csrc_v)
    pltpu.sync_copy(dst_hbm.at[pl.ds(cbase, CPT)], cdst_v)
    pltpu.sync_copy(av_hbm.at[pl.ds(cbase, CPT)], cav_v)
    pltpu.sync_copy(t_hbm.at[pl.ds(cbase, CPT)], t_v)
    pltpu.async_copy(feats_hbm.at[cdst_v], crows_v, sem).wait()
    for g in range(CPT // L):
        sl = pl.ds(g * L, L)
        dv = plsc.load_gather(denom_v, [csrc_v[sl]])
        cw_v[sl] = cav_v[sl] / dv

    def corr(j, _):
        dot = jnp.zeros((L,), jnp.float32)
        for k in range(D // L):
            sl = pl.ds(k * L, L)
            dot = dot + crows_v[j, sl] * t_v[j, sl]
        dsc = jnp.sum(dot, axis=0)
        svbc = plsc.load_gather(cw_v, [jnp.full((L,), j, jnp.int32)])
        coef = -2.0 * dsc * svbc
        for k in range(D // L):
            sl = pl.ds(k * L, L)
            crows_v[j, sl] = coef * t_v[j, sl]
        return 0

    lax.fori_loop(0, CPT, corr, 0)
    pltpu.sync_copy(crows_v, acc_sh.at[csrc_v], add=True)

    base0 = wid * EPT

    def vchunk(ci, _):
        base = base0 + ci * CK
        pltpu.sync_copy(src_hbm.at[pl.ds(base, CK)], src_v)
        pltpu.sync_copy(dst_hbm.at[pl.ds(base, CK)], dst_v)
        pltpu.sync_copy(av_hbm.at[pl.ds(base, CK)], av_v)
        gat = pltpu.async_copy(feats_hbm.at[dst_v], rows_v, sem)
        for g in range(CK // L):
            sl = pl.ds(g * L, L)
            dv = plsc.load_gather(denom_v, [src_v[sl]])
            w_v[sl] = av_v[sl] / dv
        gat.wait()
        _scale_rows(rows_v, w_v, CK)
        pltpu.sync_copy(rows_v, acc_sh.at[src_v], add=True)
        return 0

    lax.fori_loop(0, NCHUNK_V, vchunk, 0)
    plsc.subcore_barrier()
    pltpu.sync_copy(acc_sh.at[pl.ds(s * NPS, NPS)],
                    acc_out.at[c, pl.ds(s * NPS, NPS)])


NCHUNK_V = EPT // CK

_sc_pass_c = functools.partial(
    pl.kernel,
    out_type=(jax.ShapeDtypeStruct((NC, N_PAD, D), jnp.float32),
              jax.ShapeDtypeStruct((NC, N_PAD), jnp.float32)),
    mesh=_mesh,
    compiler_params=_params,
    scratch_types=[
        pltpu.VMEM((N_PAD,), jnp.float32),
        pltpu.VMEM((CK,), jnp.int32),
        pltpu.VMEM((CK,), jnp.int32),
        pltpu.VMEM((CK,), jnp.float32),
        pltpu.VMEM((CK,), jnp.float32),
        pltpu.VMEM((CK, D), jnp.float32),
        pltpu.VMEM((CPT,), jnp.int32),
        pltpu.VMEM((CPT,), jnp.int32),
        pltpu.VMEM((CPT,), jnp.float32),
        pltpu.VMEM((CPT,), jnp.float32),
        pltpu.VMEM((CPT, D), jnp.float32),
        pltpu.VMEM((CPT, D), jnp.float32),
        pltpu.VMEM_SHARED((N_PAD, D), jnp.float32),
        pltpu.VMEM_SHARED((N_PAD,), jnp.float32),
        pltpu.SemaphoreType.DMA,
    ],
)(_pass_c_body)


def _edge_ef(pa_v, qa_v, pg_v, qg_v, den1_v, src_v, dst_v, av_v, sl):
    sv16 = src_v[sl]
    dv16 = dst_v[sl]
    att = _sigmoid(plsc.load_gather(pa_v, [sv16]) + plsc.load_gather(qa_v, [dv16]))
    att = jnp.maximum(att, 1e-4)
    gate = _sigmoid(plsc.load_gather(pg_v, [sv16]) + plsc.load_gather(qg_v, [dv16]))
    sv = av_v[sl] / plsc.load_gather(den1_v, [sv16])
    final = gate * att + (1.0 - gate) * sv
    return jnp.exp(final)


def _pf_scalar_body(av_hbm, src_hbm, dst_hbm, p_hbm, den1_hbm, zn_hbm,
                    ef_out, den_out,
                    pa_v, qa_v, pg_v, qg_v, den1_v,
                    src_v, dst_v, av_v, w_v, den_sh):
    c = lax.axis_index("c")
    s = lax.axis_index("s")
    pltpu.sync_copy(zn_hbm.at[pl.ds(s * NPS, NPS)],
                    den_sh.at[pl.ds(s * NPS, NPS)])
    pltpu.sync_copy(p_hbm.at[0], pa_v)
    pltpu.sync_copy(p_hbm.at[1], qa_v)
    pltpu.sync_copy(p_hbm.at[2], pg_v)
    pltpu.sync_copy(p_hbm.at[3], qg_v)
    pltpu.sync_copy(den1_hbm, den1_v)
    plsc.subcore_barrier()

    sbase0 = s * EPSC

    def schunk(ci, _):
        base = sbase0 + ci * CK
        pltpu.sync_copy(src_hbm.at[pl.ds(base, CK)], src_v)
        pltpu.sync_copy(dst_hbm.at[pl.ds(base, CK)], dst_v)
        pltpu.sync_copy(av_hbm.at[pl.ds(base, CK)], av_v)
        for g in range(CK // L):
            sl = pl.ds(g * L, L)
            w_v[sl] = _edge_ef(pa_v, qa_v, pg_v, qg_v, den1_v,
                               src_v, dst_v, av_v, sl)
        pltpu.sync_copy(w_v, den_sh.at[src_v], add=True)
        pltpu.sync_copy(w_v, ef_out.at[c, pl.ds(base, CK)])
        return 0

    lax.fori_loop(0, EPSC // CK, schunk, 0)
    plsc.subcore_barrier()
    pltpu.sync_copy(den_sh.at[pl.ds(s * NPS, NPS)],
                    den_out.at[c, pl.ds(s * NPS, NPS)])


_sc_pf_scalar = functools.partial(
    pl.kernel,
    out_type=(jax.ShapeDtypeStruct((NC, E_PAD), jnp.float32),
              jax.ShapeDtypeStruct((NC, N_PAD), jnp.float32)),
    mesh=_mesh,
    compiler_params=_params,
    scratch_types=[
        pltpu.VMEM((N_PAD,), jnp.float32),
        pltpu.VMEM((N_PAD,), jnp.float32),
        pltpu.VMEM((N_PAD,), jnp.float32),
        pltpu.VMEM((N_PAD,), jnp.float32),
        pltpu.VMEM((N_PAD,), jnp.float32),
        pltpu.VMEM((CK,), jnp.int32),
        pltpu.VMEM((CK,), jnp.int32),
        pltpu.VMEM((CK,), jnp.float32),
        pltpu.VMEM((CK,), jnp.float32),
        pltpu.VMEM_SHARED((N_PAD,), jnp.float32),
    ],
)(_pf_scalar_body)


def _pf_vector_body(feats_hbm, ef_hbm, src_hbm, dst_hbm, den2_hbm, zrows_hbm,
                    acc_out,
                    den2_v, src_v, dst_v, av_v, w_v, rows_v, acc_sh, sem):
    c = lax.axis_index("c")
    s = lax.axis_index("s")
    wid = c * NS + s
    pltpu.sync_copy(zrows_hbm.at[pl.ds(s * NPS, NPS)],
                    acc_sh.at[pl.ds(s * NPS, NPS)])
    pltpu.sync_copy(den2_hbm, den2_v)
    plsc.subcore_barrier()
    base0 = wid * EPT

    def vchunk(ci, _):
        base = base0 + ci * CK
        pltpu.sync_copy(src_hbm.at[pl.ds(base, CK)], src_v)
        pltpu.sync_copy(dst_hbm.at[pl.ds(base, CK)], dst_v)
        pltpu.sync_copy(ef_hbm.at[c, pl.ds(base, CK)], av_v)
        gat = pltpu.async_copy(feats_hbm.at[dst_v], rows_v, sem)
        for g in range(CK // L):
            sl = pl.ds(g * L, L)
            w_v[sl] = av_v[sl] / plsc.load_gather(den2_v, [src_v[sl]])
        gat.wait()
        _scale_rows(rows_v, w_v, CK)
        pltpu.sync_copy(rows_v, acc_sh.at[src_v], add=True)
        return 0

    lax.fori_loop(0, NCHUNK_V, vchunk, 0)
    plsc.subcore_barrier()
    pltpu.sync_copy(acc_sh.at[pl.ds(s * NPS, NPS)],
                    acc_out.at[c, pl.ds(s * NPS, NPS)])


_sc_pf_vector = functools.partial(
    pl.kernel,
    out_type=jax.ShapeDtypeStruct((NC, N_PAD, D), jnp.float32),
    mesh=_mesh,
    compiler_params=_params,
    scratch_types=[
        pltpu.VMEM((N_PAD,), jnp.float32),
        pltpu.VMEM((CK,), jnp.int32),
        pltpu.VMEM((CK,), jnp.int32),
        pltpu.VMEM((CK,), jnp.float32),
        pltpu.VMEM((CK,), jnp.float32),
        pltpu.VMEM((CK, D), jnp.float32),
        pltpu.VMEM_SHARED((N_PAD, D), jnp.float32),
        pltpu.SemaphoreType.DMA,
    ],
)(_pf_vector_body)


def _tanh_body(x_ref, o_ref):
    o_ref[...] = jnp.tanh(x_ref[...])


def _tc_tanh(x):
    n = x.shape[0]
    blk = 1000
    return pl.pallas_call(
        _tanh_body,
        grid=(n // blk,),
        in_specs=[pl.BlockSpec((blk, D), lambda i: (i, 0))],
        out_specs=pl.BlockSpec((blk, D), lambda i: (i, 0)),
        out_shape=jax.ShapeDtypeStruct((n, D), jnp.float32),
    )(x)


def kernel(features, rel_emb, adj, r_index, r_val, k0, k1, W_attn, b_attn, W_gate, b_gate):
    f32 = jnp.float32
    src, dst = adj[0], adj[1]
    pad_i = jnp.full((E_PAD - E,), PAD_SRC, jnp.int32)
    pad_z = jnp.zeros((E_PAD - E,), jnp.int32)
    src_p = jnp.concatenate([src, pad_i])
    dst_p = jnp.concatenate([dst, pad_z])
    r0_p = jnp.concatenate([r_index[0], pad_z])
    r1_p = jnp.concatenate([r_index[1], pad_z])
    rv_p = jnp.concatenate([r_val, jnp.zeros((E_PAD - E,), f32)])
    zeros_nd = jnp.zeros((N_PAD, D), f32)
    zeros_n = jnp.zeros((N_PAD,), f32)

    feats = _tc_tanh(features)
    gparts = _sc_phase_a(rel_emb, r0_p, r1_p, rv_p, zeros_nd[:R_PAD])
    g = (gparts[0] + gparts[1])[:R]
    nrm = jnp.sqrt(jnp.sum(g * g, axis=1, keepdims=True))
    t = g / jnp.maximum(nrm, 1e-12)
    t_pad = jnp.concatenate([t, jnp.zeros((R_PAD - R, D), f32)], axis=0)
    att_s = t @ jnp.concatenate([k0, k1], axis=1)
    wa = W_attn[0]
    wg = W_gate[0]
    Wp = jnp.stack([wa[:D] + wa[2 * D:], wa[D:2 * D] - wa[2 * D:],
                    wg[:D] + wg[2 * D:], wg[D:2 * D] - wg[2 * D:]], axis=1)
    ba = jnp.stack([b_attn[0] * 0.5, b_attn[0] * 0.5, b_gate[0] * 0.5, b_gate[0] * 0.5])

    outc = [feats]
    outs = [feats]
    for l in range(DEPTH):
        av = jnp.concatenate([jnp.exp(att_s[:, l]),
                              jnp.ones((E - R,), f32),
                              jnp.zeros((E_PAD - E,), f32)])
        accp, den1p = _sc_pass_c(feats, av, src_p, dst_p, t_pad,
                                 zeros_n, zeros_nd)
        feats = _tc_tanh((accp[0] + accp[1])[:N])
        outc.append(feats)
        denom1 = den1p[0]
        P = feats @ Wp + ba[None, :]
        P_pad = jnp.concatenate([P, jnp.zeros((N_PAD - N, 4), f32)], axis=0).T
        ef2, den2p = _sc_pf_scalar(av, src_p, dst_p, P_pad, denom1, zeros_n)
        acc2p = _sc_pf_vector(feats, ef2, src_p, dst_p, den2p[0], zeros_nd)
        acc2 = (acc2p[0] + acc2p[1])[:N]
        s1 = jnp.where(den2p[0][:N] > 0, 1.0, 0.0)
        outs.append(_tc_tanh(feats * s1[:, None] - acc2))
    return (jnp.concatenate(outc, axis=-1), jnp.concatenate(outs, axis=-1))


# speedup vs baseline: 19.9504x; 1.5996x over previous
import functools
import jax
import jax.numpy as jnp
from jax import lax
from jax.experimental import pallas as pl
from jax.experimental.pallas import tpu as pltpu
from jax.experimental.pallas import tpu_sc as plsc

N = 10000
E = 320000
R = 1000
D = 128
DEPTH = 2

NC = 2
NS = 16
L = 16

CK = 128
EPT = 10112
E_PAD = EPT * NC * NS
EPSC = E_PAD // NS
NCHUNK_V = EPT // CK
NCHUNK_S = EPSC // CK
N_PAD = 10240
NPS = N_PAD // NS
R_PAD = 1024
RPS = R_PAD // NS
CPT = R_PAD // (NC * NS)
PAD_SRC = N_PAD - 1

_mesh = plsc.VectorSubcoreMesh(core_axis_name="c", subcore_axis_name="s",
                               num_cores=NC, num_subcores=NS)
_params = pltpu.CompilerParams(needs_layout_passes=False)


def _sigmoid(x):
    return 1.0 / (1.0 + jnp.exp(-x))


def _scale_rows(rows_v, w_v, nrows):
    def body(j, _):
        wbc = plsc.load_gather(w_v, [jnp.full((L,), j, jnp.int32)])
        for k in range(D // L):
            sl = pl.ds(k * L, L)
            rows_v[j, sl] = rows_v[j, sl] * wbc
        return 0

    lax.fori_loop(0, nrows, body, 0)


def _pipe_vector_loop(base0, nchunk, sets, load_idx, compute_w, feats_hbm,
                      acc_sh):
    def start_gather(st):
        pltpu.async_copy(feats_hbm.at[st[1]], st[3], st[4])

    def wait_gather(st):
        pltpu.make_async_copy(feats_hbm.at[st[1]], st[3], st[4]).wait()

    def finish(st):
        compute_w(st)
        wait_gather(st)
        _scale_rows(st[3], st[2], CK)
        pltpu.sync_copy(st[3], acc_sh.at[st[0]], add=True)

    load_idx(sets[0], base0)
    start_gather(sets[0])

    def dbl(ii, _):
        for p in (0, 1):
            i = 2 * ii + p
            q = 1 - p
            load_idx(sets[q], base0 + (i + 1) * CK)
            start_gather(sets[q])
            finish(sets[p])
        return 0

    lax.fori_loop(0, (nchunk - 1) // 2, dbl, 0)
    finish(sets[(nchunk - 1) % 2])


def _phase_a_body(rel_hbm, r0_hbm, r1_hbm, rv_hbm, zrows_hbm, out_hbm,
                  i0a, i1a, wa, rowsa, i0b, i1b, wb, rowsb,
                  acc_sh, sema, semb):
    c = lax.axis_index("c")
    s = lax.axis_index("s")
    wid = c * NS + s
    pltpu.sync_copy(zrows_hbm.at[pl.ds(s * RPS, RPS)],
                    acc_sh.at[pl.ds(s * RPS, RPS)])
    plsc.subcore_barrier()

    def load_idx(st, base):
        pltpu.sync_copy(r0_hbm.at[pl.ds(base, CK)], st[0])
        pltpu.sync_copy(r1_hbm.at[pl.ds(base, CK)], st[1])
        pltpu.sync_copy(rv_hbm.at[pl.ds(base, CK)], st[2])

    sets = ((i0a, i1a, wa, rowsa, sema), (i0b, i1b, wb, rowsb, semb))
    _pipe_vector_loop(wid * EPT, NCHUNK_V, sets, load_idx, lambda st: None,
                      rel_hbm, acc_sh)
    plsc.subcore_barrier()
    pltpu.sync_copy(acc_sh.at[pl.ds(s * RPS, RPS)],
                    out_hbm.at[c, pl.ds(s * RPS, RPS)])


_sc_phase_a = functools.partial(
    pl.kernel,
    out_type=jax.ShapeDtypeStruct((NC, R_PAD, D), jnp.float32),
    mesh=_mesh,
    compiler_params=_params,
    scratch_types=[
        pltpu.VMEM((CK,), jnp.int32),
        pltpu.VMEM((CK,), jnp.int32),
        pltpu.VMEM((CK,), jnp.float32),
        pltpu.VMEM((CK, D), jnp.float32),
        pltpu.VMEM((CK,), jnp.int32),
        pltpu.VMEM((CK,), jnp.int32),
        pltpu.VMEM((CK,), jnp.float32),
        pltpu.VMEM((CK, D), jnp.float32),
        pltpu.VMEM_SHARED((R_PAD, D), jnp.float32),
        pltpu.SemaphoreType.DMA,
        pltpu.SemaphoreType.DMA,
    ],
)(_phase_a_body)


def _pass_c_body(feats_hbm, av_hbm, src_hbm, dst_hbm, t_hbm,
                 zn_hbm, zrows_hbm, acc_out, den_out,
                 denom_v, srca, dsta, ava, wa, rowsa, srcb, dstb, avb, wb,
                 rowsb, csrc_v, cdst_v, cav_v, cw_v,
                 acc_sh, den_sh, sema, semb):
    c = lax.axis_index("c")
    s = lax.axis_index("s")
    wid = c * NS + s
    pltpu.sync_copy(zrows_hbm.at[pl.ds(s * NPS, NPS)],
                    acc_sh.at[pl.ds(s * NPS, NPS)])
    pltpu.sync_copy(zn_hbm.at[pl.ds(s * NPS, NPS)],
                    den_sh.at[pl.ds(s * NPS, NPS)])
    plsc.subcore_barrier()

    sbase0 = s * EPSC

    def sload(st, base):
        pltpu.async_copy(src_hbm.at[pl.ds(base, CK)], st[0], st[4])
        pltpu.async_copy(av_hbm.at[pl.ds(base, CK)], st[2], st[4])

    def swait(st, base):
        pltpu.make_async_copy(src_hbm.at[pl.ds(base, CK)], st[0], st[4]).wait()
        pltpu.make_async_copy(av_hbm.at[pl.ds(base, CK)], st[2], st[4]).wait()

    ssets = ((srca, dsta, ava, wa, sema), (srcb, dstb, avb, wb, semb))
    sload(ssets[0], sbase0)

    def sdbl(ii, _):
        for p in (0, 1):
            i = 2 * ii + p
            q = 1 - p
            sload(ssets[q], sbase0 + (i + 1) * CK)
            swait(ssets[p], sbase0 + i * CK)
            pltpu.sync_copy(ssets[p][2], den_sh.at[ssets[p][0]], add=True)
        return 0

    lax.fori_loop(0, NCHUNK_S // 2 - 1, sdbl, 0)
    sload(ssets[1], sbase0 + (NCHUNK_S - 1) * CK)
    swait(ssets[0], sbase0 + (NCHUNK_S - 2) * CK)
    pltpu.sync_copy(ssets[0][2], den_sh.at[ssets[0][0]], add=True)
    swait(ssets[1], sbase0 + (NCHUNK_S - 1) * CK)
    pltpu.sync_copy(ssets[1][2], den_sh.at[ssets[1][0]], add=True)
    plsc.subcore_barrier()
    pltpu.sync_copy(den_sh, denom_v)
    pltpu.sync_copy(den_sh.at[pl.ds(s * NPS, NPS)],
                    den_out.at[c, pl.ds(s * NPS, NPS)])

    cbase = wid * CPT
    pltpu.sync_copy(src_hbm.at[pl.ds(cbase, CPT)], csrc_v)
    pltpu.sync_copy(dst_hbm.at[pl.ds(cbase, CPT)], cdst_v)
    pltpu.sync_copy(av_hbm.at[pl.ds(cbase, CPT)], cav_v)
    pltpu.sync_copy(t_hbm.at[pl.ds(cbase, CPT)], rowsb.at[pl.ds(0, CPT)])
    pltpu.async_copy(feats_hbm.at[cdst_v], rowsa.at[pl.ds(0, CPT)], sema).wait()
    for g in range(CPT // L):
        sl = pl.ds(g * L, L)
        dv = plsc.load_gather(denom_v, [csrc_v[sl]])
        cw_v[sl] = cav_v[sl] / dv

    def corr(j, _):
        dot = jnp.zeros((L,), jnp.float32)
        for k in range(D // L):
            sl = pl.ds(k * L, L)
            dot = dot + rowsa[j, sl] * rowsb[j, sl]
        dsc = jnp.sum(dot, axis=0)
        svbc = plsc.load_gather(cw_v, [jnp.full((L,), j, jnp.int32)])
        coef = -2.0 * dsc * svbc
        for k in range(D // L):
            sl = pl.ds(k * L, L)
            rowsa[j, sl] = coef * rowsb[j, sl]
        return 0

    lax.fori_loop(0, CPT, corr, 0)
    pltpu.sync_copy(rowsa.at[pl.ds(0, CPT)], acc_sh.at[csrc_v], add=True)

    def vload(st, base):
        pltpu.sync_copy(src_hbm.at[pl.ds(base, CK)], st[0])
        pltpu.sync_copy(dst_hbm.at[pl.ds(base, CK)], st[1])
        pltpu.sync_copy(av_hbm.at[pl.ds(base, CK)], st[5])

    def vcompw(st):
        for g in range(CK // L):
            sl = pl.ds(g * L, L)
            dv = plsc.load_gather(denom_v, [st[0][sl]])
            st[2][sl] = st[5][sl] / dv

    vsets = ((srca, dsta, wa, rowsa, sema, ava),
             (srcb, dstb, wb, rowsb, semb, avb))
    _pipe_vector_loop(wid * EPT, NCHUNK_V, vsets, vload, vcompw,
                      feats_hbm, acc_sh)
    plsc.subcore_barrier()
    pltpu.sync_copy(acc_sh.at[pl.ds(s * NPS, NPS)],
                    acc_out.at[c, pl.ds(s * NPS, NPS)])


_sc_pass_c = functools.partial(
    pl.kernel,
    out_type=(jax.ShapeDtypeStruct((NC, N_PAD, D), jnp.float32),
              jax.ShapeDtypeStruct((NC, N_PAD), jnp.float32)),
    mesh=_mesh,
    compiler_params=_params,
    scratch_types=[
        pltpu.VMEM((N_PAD,), jnp.float32),
        pltpu.VMEM((CK,), jnp.int32),
        pltpu.VMEM((CK,), jnp.int32),
        pltpu.VMEM((CK,), jnp.float32),
        pltpu.VMEM((CK,), jnp.float32),
        pltpu.VMEM((CK, D), jnp.float32),
        pltpu.VMEM((CK,), jnp.int32),
        pltpu.VMEM((CK,), jnp.int32),
        pltpu.VMEM((CK,), jnp.float32),
        pltpu.VMEM((CK,), jnp.float32),
        pltpu.VMEM((CK, D), jnp.float32),
        pltpu.VMEM((CPT,), jnp.int32),
        pltpu.VMEM((CPT,), jnp.int32),
        pltpu.VMEM((CPT,), jnp.float32),
        pltpu.VMEM((CPT,), jnp.float32),
        pltpu.VMEM_SHARED((N_PAD, D), jnp.float32),
        pltpu.VMEM_SHARED((N_PAD,), jnp.float32),
        pltpu.SemaphoreType.DMA,
        pltpu.SemaphoreType.DMA,
    ],
)(_pass_c_body)


def _edge_ef(pa_v, qa_v, pg_v, qg_v, den1_v, src_v, dst_v, av_v, sl):
    sv16 = src_v[sl]
    dv16 = dst_v[sl]
    att = _sigmoid(plsc.load_gather(pa_v, [sv16]) + plsc.load_gather(qa_v, [dv16]))
    att = jnp.maximum(att, 1e-4)
    gate = _sigmoid(plsc.load_gather(pg_v, [sv16]) + plsc.load_gather(qg_v, [dv16]))
    sv = av_v[sl] / plsc.load_gather(den1_v, [sv16])
    final = gate * att + (1.0 - gate) * sv
    return jnp.exp(final)


def _pf_scalar_body(av_hbm, src_hbm, dst_hbm, p_hbm, den1_hbm, zn_hbm,
                    ef_out, den_out,
                    pa_v, qa_v, pg_v, qg_v, den1_v,
                    srca, dsta, ava, wa, srcb, dstb, avb, wb,
                    den_sh, sema, semb):
    c = lax.axis_index("c")
    s = lax.axis_index("s")
    pltpu.sync_copy(zn_hbm.at[pl.ds(s * NPS, NPS)],
                    den_sh.at[pl.ds(s * NPS, NPS)])
    pltpu.sync_copy(p_hbm.at[0], pa_v)
    pltpu.sync_copy(p_hbm.at[1], qa_v)
    pltpu.sync_copy(p_hbm.at[2], pg_v)
    pltpu.sync_copy(p_hbm.at[3], qg_v)
    pltpu.sync_copy(den1_hbm, den1_v)
    plsc.subcore_barrier()

    sbase0 = s * EPSC

    def sload(st, base):
        pltpu.async_copy(src_hbm.at[pl.ds(base, CK)], st[0], st[4])
        pltpu.async_copy(dst_hbm.at[pl.ds(base, CK)], st[1], st[4])
        pltpu.async_copy(av_hbm.at[pl.ds(base, CK)], st[2], st[4])

    def swait(st, base):
        pltpu.make_async_copy(src_hbm.at[pl.ds(base, CK)], st[0], st[4]).wait()
        pltpu.make_async_copy(dst_hbm.at[pl.ds(base, CK)], st[1], st[4]).wait()
        pltpu.make_async_copy(av_hbm.at[pl.ds(base, CK)], st[2], st[4]).wait()

    def sfin(st, base):
        for g in range(CK // L):
            sl = pl.ds(g * L, L)
            st[3][sl] = _edge_ef(pa_v, qa_v, pg_v, qg_v, den1_v,
                                 st[0], st[1], st[2], sl)
        pltpu.sync_copy(st[3], den_sh.at[st[0]], add=True)
        pltpu.sync_copy(st[3], ef_out.at[c, pl.ds(base, CK)])

    ssets = ((srca, dsta, ava, wa, sema), (srcb, dstb, avb, wb, semb))
    sload(ssets[0], sbase0)

    def sdbl(ii, _):
        for p in (0, 1):
            i = 2 * ii + p
            q = 1 - p
            sload(ssets[q], sbase0 + (i + 1) * CK)
            swait(ssets[p], sbase0 + i * CK)
            sfin(ssets[p], sbase0 + i * CK)
        return 0

    lax.fori_loop(0, NCHUNK_S // 2 - 1, sdbl, 0)
    sload(ssets[1], sbase0 + (NCHUNK_S - 1) * CK)
    swait(ssets[0], sbase0 + (NCHUNK_S - 2) * CK)
    sfin(ssets[0], sbase0 + (NCHUNK_S - 2) * CK)
    swait(ssets[1], sbase0 + (NCHUNK_S - 1) * CK)
    sfin(ssets[1], sbase0 + (NCHUNK_S - 1) * CK)
    plsc.subcore_barrier()
    pltpu.sync_copy(den_sh.at[pl.ds(s * NPS, NPS)],
                    den_out.at[c, pl.ds(s * NPS, NPS)])


_sc_pf_scalar = functools.partial(
    pl.kernel,
    out_type=(jax.ShapeDtypeStruct((NC, E_PAD), jnp.float32),
              jax.ShapeDtypeStruct((NC, N_PAD), jnp.float32)),
    mesh=_mesh,
    compiler_params=_params,
    scratch_types=[
        pltpu.VMEM((N_PAD,), jnp.float32),
        pltpu.VMEM((N_PAD,), jnp.float32),
        pltpu.VMEM((N_PAD,), jnp.float32),
        pltpu.VMEM((N_PAD,), jnp.float32),
        pltpu.VMEM((N_PAD,), jnp.float32),
        pltpu.VMEM((CK,), jnp.int32),
        pltpu.VMEM((CK,), jnp.int32),
        pltpu.VMEM((CK,), jnp.float32),
        pltpu.VMEM((CK,), jnp.float32),
        pltpu.VMEM((CK,), jnp.int32),
        pltpu.VMEM((CK,), jnp.int32),
        pltpu.VMEM((CK,), jnp.float32),
        pltpu.VMEM((CK,), jnp.float32),
        pltpu.VMEM_SHARED((N_PAD,), jnp.float32),
        pltpu.SemaphoreType.DMA,
        pltpu.SemaphoreType.DMA,
    ],
)(_pf_scalar_body)


def _pf_vector_body(feats_hbm, ef_hbm, src_hbm, dst_hbm, den2_hbm, zrows_hbm,
                    acc_out,
                    den2_v, srca, dsta, ava, wa, rowsa, srcb, dstb, avb, wb,
                    rowsb, acc_sh, sema, semb):
    c = lax.axis_index("c")
    s = lax.axis_index("s")
    wid = c * NS + s
    pltpu.sync_copy(zrows_hbm.at[pl.ds(s * NPS, NPS)],
                    acc_sh.at[pl.ds(s * NPS, NPS)])
    pltpu.sync_copy(den2_hbm, den2_v)
    plsc.subcore_barrier()

    def vload(st, base):
        pltpu.sync_copy(src_hbm.at[pl.ds(base, CK)], st[0])
        pltpu.sync_copy(dst_hbm.at[pl.ds(base, CK)], st[1])
        pltpu.sync_copy(ef_hbm.at[c, pl.ds(base, CK)], st[5])

    def vcompw(st):
        for g in range(CK // L):
            sl = pl.ds(g * L, L)
            dv = plsc.load_gather(den2_v, [st[0][sl]])
            st[2][sl] = st[5][sl] / dv

    vsets = ((srca, dsta, wa, rowsa, sema, ava),
             (srcb, dstb, wb, rowsb, semb, avb))
    _pipe_vector_loop(wid * EPT, NCHUNK_V, vsets, vload, vcompw,
                      feats_hbm, acc_sh)
    plsc.subcore_barrier()
    pltpu.sync_copy(acc_sh.at[pl.ds(s * NPS, NPS)],
                    acc_out.at[c, pl.ds(s * NPS, NPS)])


_sc_pf_vector = functools.partial(
    pl.kernel,
    out_type=jax.ShapeDtypeStruct((NC, N_PAD, D), jnp.float32),
    mesh=_mesh,
    compiler_params=_params,
    scratch_types=[
        pltpu.VMEM((N_PAD,), jnp.float32),
        pltpu.VMEM((CK,), jnp.int32),
        pltpu.VMEM((CK,), jnp.int32),
        pltpu.VMEM((CK,), jnp.float32),
        pltpu.VMEM((CK,), jnp.float32),
        pltpu.VMEM((CK, D), jnp.float32),
        pltpu.VMEM((CK,), jnp.int32),
        pltpu.VMEM((CK,), jnp.int32),
        pltpu.VMEM((CK,), jnp.float32),
        pltpu.VMEM((CK,), jnp.float32),
        pltpu.VMEM((CK, D), jnp.float32),
        pltpu.VMEM_SHARED((N_PAD, D), jnp.float32),
        pltpu.SemaphoreType.DMA,
        pltpu.SemaphoreType.DMA,
    ],
)(_pf_vector_body)


def _tanh_body(x_ref, o_ref):
    o_ref[...] = jnp.tanh(x_ref[...])


def _tc_tanh(x):
    n = x.shape[0]
    blk = 1000
    return pl.pallas_call(
        _tanh_body,
        grid=(n // blk,),
        in_specs=[pl.BlockSpec((blk, D), lambda i: (i, 0))],
        out_specs=pl.BlockSpec((blk, D), lambda i: (i, 0)),
        out_shape=jax.ShapeDtypeStruct((n, D), jnp.float32),
    )(x)


def kernel(features, rel_emb, adj, r_index, r_val, k0, k1, W_attn, b_attn, W_gate, b_gate):
    f32 = jnp.float32
    src, dst = adj[0], adj[1]
    pad_i = jnp.full((E_PAD - E,), PAD_SRC, jnp.int32)
    pad_z = jnp.zeros((E_PAD - E,), jnp.int32)
    src_p = jnp.concatenate([src, pad_i])
    dst_p = jnp.concatenate([dst, pad_z])
    r0_p = jnp.concatenate([r_index[0], pad_z])
    r1_p = jnp.concatenate([r_index[1], pad_z])
    rv_p = jnp.concatenate([r_val, jnp.zeros((E_PAD - E,), f32)])
    zeros_nd = jnp.zeros((N_PAD, D), f32)
    zeros_n = jnp.zeros((N_PAD,), f32)

    feats = _tc_tanh(features)
    gparts = _sc_phase_a(rel_emb, r0_p, r1_p, rv_p, zeros_nd[:R_PAD])
    g = (gparts[0] + gparts[1])[:R]
    nrm = jnp.sqrt(jnp.sum(g * g, axis=1, keepdims=True))
    t = g / jnp.maximum(nrm, 1e-12)
    t_pad = jnp.concatenate([t, jnp.zeros((R_PAD - R, D), f32)], axis=0)
    att_s = t @ jnp.concatenate([k0, k1], axis=1)
    wa = W_attn[0]
    wg = W_gate[0]
    Wp = jnp.stack([wa[:D] + wa[2 * D:], wa[D:2 * D] - wa[2 * D:],
                    wg[:D] + wg[2 * D:], wg[D:2 * D] - wg[2 * D:]], axis=1)
    ba = jnp.stack([b_attn[0] * 0.5, b_attn[0] * 0.5, b_gate[0] * 0.5, b_gate[0] * 0.5])

    outc = [feats]
    outs = [feats]
    for l in range(DEPTH):
        av = jnp.concatenate([jnp.exp(att_s[:, l]),
                              jnp.ones((E - R,), f32),
                              jnp.zeros((E_PAD - E,), f32)])
        accp, den1p = _sc_pass_c(feats, av, src_p, dst_p, t_pad,
                                 zeros_n, zeros_nd)
        feats = _tc_tanh((accp[0] + accp[1])[:N])
        outc.append(feats)
        denom1 = den1p[0]
        P = feats @ Wp + ba[None, :]
        P_pad = jnp.concatenate([P, jnp.zeros((N_PAD - N, 4), f32)], axis=0).T
        ef2, den2p = _sc_pf_scalar(av, src_p, dst_p, P_pad, denom1, zeros_n)
        acc2p = _sc_pf_vector(feats, ef2, src_p, dst_p, den2p[0], zeros_nd)
        acc2 = (acc2p[0] + acc2p[1])[:N]
        s1 = jnp.where(den2p[0][:N] > 0, 1.0, 0.0)
        outs.append(_tc_tanh(feats * s1[:, None] - acc2))
    return (jnp.concatenate(outc, axis=-1), jnp.concatenate(outs, axis=-1))


# speedup vs baseline: 21.3859x; 1.0720x over previous
import functools
import jax
import jax.numpy as jnp
from jax import lax
from jax.experimental import pallas as pl
from jax.experimental.pallas import tpu as pltpu
from jax.experimental.pallas import tpu_sc as plsc

N = 10000
E = 320000
R = 1000
D = 128
DEPTH = 2

NC = 2
NS = 16
L = 16

CK = 128
EPT = 10112
E_PAD = EPT * NC * NS
E_PAD2 = E_PAD + CK
NCH2 = E_PAD2 // CK
EPSC = E_PAD // NS
NCHUNK_V = EPT // CK
NCHUNK_S = EPSC // CK
N_PAD = 10240
NPS = N_PAD // NS
R_PAD = 1024
RPS = R_PAD // NS
CPT = R_PAD // (NC * NS)
PAD_SRC = N_PAD - 1

_mesh = plsc.VectorSubcoreMesh(core_axis_name="c", subcore_axis_name="s",
                               num_cores=NC, num_subcores=NS)
_params = pltpu.CompilerParams(needs_layout_passes=False)


def _sigmoid(x):
    return 1.0 / (1.0 + jnp.exp(-x))


def _scale_rows(rows_v, w_v, nrows):
    def body(j, _):
        wbc = plsc.load_gather(w_v, [jnp.full((L,), j, jnp.int32)])
        for k in range(D // L):
            sl = pl.ds(k * L, L)
            rows_v[j, sl] = rows_v[j, sl] * wbc
        return 0

    lax.fori_loop(0, nrows, body, 0)


def _pipe_vector_loop(cid0, nchunk, sets, load_idx, compute_w, feats_hbm,
                      acc_sh):
    def start_gather(st):
        pltpu.async_copy(feats_hbm.at[st[0].at[1]], st[2], st[3])

    def wait_gather(st):
        pltpu.make_async_copy(feats_hbm.at[st[0].at[1]], st[2], st[3]).wait()

    def finish(st):
        compute_w(st)
        wait_gather(st)
        _scale_rows(st[2], st[1], CK)
        pltpu.sync_copy(st[2], acc_sh.at[st[0].at[0]], add=True)

    load_idx(sets[0], cid0)
    start_gather(sets[0])

    def dbl(ii, _):
        for p in (0, 1):
            i = 2 * ii + p
            q = 1 - p
            load_idx(sets[q], cid0 + i + 1)
            start_gather(sets[q])
            finish(sets[p])
        return 0

    lax.fori_loop(0, (nchunk - 1) // 2, dbl, 0)
    finish(sets[(nchunk - 1) % 2])


def _phase_a_body(rel_hbm, pk_hbm, zrows_hbm, out_hbm,
                  pka, wa, rowsa, pkb, wb, rowsb, acc_sh, sema, semb):
    c = lax.axis_index("c")
    s = lax.axis_index("s")
    wid = c * NS + s
    pltpu.sync_copy(zrows_hbm.at[pl.ds(s * RPS, RPS)],
                    acc_sh.at[pl.ds(s * RPS, RPS)])
    plsc.subcore_barrier()

    def load_idx(st, cid):
        pltpu.sync_copy(pk_hbm.at[cid], st[0])

    def compw(st):
        for g in range(CK // L):
            sl = pl.ds(g * L, L)
            st[1][sl] = plsc.bitcast(st[0][2, sl], jnp.float32)

    sets = ((pka, wa, rowsa, sema), (pkb, wb, rowsb, semb))
    _pipe_vector_loop(wid * NCHUNK_V, NCHUNK_V, sets, load_idx, compw,
                      rel_hbm, acc_sh)
    plsc.subcore_barrier()
    pltpu.sync_copy(acc_sh.at[pl.ds(s * RPS, RPS)],
                    out_hbm.at[c, pl.ds(s * RPS, RPS)])


_sc_phase_a = functools.partial(
    pl.kernel,
    out_type=jax.ShapeDtypeStruct((NC, R_PAD, D), jnp.float32),
    mesh=_mesh,
    compiler_params=_params,
    scratch_types=[
        pltpu.VMEM((3, CK), jnp.int32),
        pltpu.VMEM((CK,), jnp.float32),
        pltpu.VMEM((CK, D), jnp.float32),
        pltpu.VMEM((3, CK), jnp.int32),
        pltpu.VMEM((CK,), jnp.float32),
        pltpu.VMEM((CK, D), jnp.float32),
        pltpu.VMEM_SHARED((R_PAD, D), jnp.float32),
        pltpu.SemaphoreType.DMA,
        pltpu.SemaphoreType.DMA,
    ],
)(_phase_a_body)


def _pass_c_body(feats_hbm, pk_hbm, src_hbm, dst_hbm, av_hbm, t_hbm,
                 zn_hbm, zrows_hbm, acc_out, den_out,
                 denom_v, pka, wa, rowsa, pkb, wb, rowsb,
                 csrc_v, cdst_v, cav_v, cw_v,
                 acc_sh, den_sh, sema, semb):
    c = lax.axis_index("c")
    s = lax.axis_index("s")
    wid = c * NS + s
    pltpu.sync_copy(zrows_hbm.at[pl.ds(s * NPS, NPS)],
                    acc_sh.at[pl.ds(s * NPS, NPS)])
    pltpu.sync_copy(zn_hbm.at[pl.ds(s * NPS, NPS)],
                    den_sh.at[pl.ds(s * NPS, NPS)])
    plsc.subcore_barrier()

    scid0 = s * NCHUNK_S

    def sload(st, cid):
        pltpu.async_copy(pk_hbm.at[cid], st[0], st[2])

    def swait(st, cid):
        pltpu.make_async_copy(pk_hbm.at[cid], st[0], st[2]).wait()

    def sfin(st):
        for g in range(CK // L):
            sl = pl.ds(g * L, L)
            st[1][sl] = plsc.bitcast(st[0][2, sl], jnp.float32)
        pltpu.sync_copy(st[1], den_sh.at[st[0].at[0]], add=True)

    ssets = ((pka, wa, sema), (pkb, wb, semb))
    sload(ssets[0], scid0)

    def sdbl(ii, _):
        for p in (0, 1):
            i = 2 * ii + p
            q = 1 - p
            sload(ssets[q], scid0 + i + 1)
            swait(ssets[p], scid0 + i)
            sfin(ssets[p])
        return 0

    lax.fori_loop(0, NCHUNK_S // 2 - 1, sdbl, 0)
    sload(ssets[1], scid0 + NCHUNK_S - 1)
    swait(ssets[0], scid0 + NCHUNK_S - 2)
    sfin(ssets[0])
    swait(ssets[1], scid0 + NCHUNK_S - 1)
    sfin(ssets[1])
    plsc.subcore_barrier()
    pltpu.sync_copy(den_sh, denom_v)
    pltpu.sync_copy(den_sh.at[pl.ds(s * NPS, NPS)],
                    den_out.at[c, pl.ds(s * NPS, NPS)])

    cbase = wid * CPT
    pltpu.sync_copy(src_hbm.at[pl.ds(cbase, CPT)], csrc_v)
    pltpu.sync_copy(dst_hbm.at[pl.ds(cbase, CPT)], cdst_v)
    pltpu.sync_copy(av_hbm.at[pl.ds(cbase, CPT)], cav_v)
    pltpu.sync_copy(t_hbm.at[pl.ds(cbase, CPT)], rowsb.at[pl.ds(0, CPT)])
    pltpu.async_copy(feats_hbm.at[cdst_v], rowsa.at[pl.ds(0, CPT)], sema).wait()
    for g in range(CPT // L):
        sl = pl.ds(g * L, L)
        dv = plsc.load_gather(denom_v, [csrc_v[sl]])
        cw_v[sl] = cav_v[sl] / dv

    def corr(j, _):
        dot = jnp.zeros((L,), jnp.float32)
        for k in range(D // L):
            sl = pl.ds(k * L, L)
            dot = dot + rowsa[j, sl] * rowsb[j, sl]
        dsc = jnp.sum(dot, axis=0)
        svbc = plsc.load_gather(cw_v, [jnp.full((L,), j, jnp.int32)])
        coef = -2.0 * dsc * svbc
        for k in range(D // L):
            sl = pl.ds(k * L, L)
            rowsa[j, sl] = coef * rowsb[j, sl]
        return 0

    lax.fori_loop(0, CPT, corr, 0)
    pltpu.sync_copy(rowsa.at[pl.ds(0, CPT)], acc_sh.at[csrc_v], add=True)

    def vload(st, cid):
        pltpu.sync_copy(pk_hbm.at[cid], st[0])

    def vcompw(st):
        for g in range(CK // L):
            sl = pl.ds(g * L, L)
            dv = plsc.load_gather(denom_v, [st[0][0, sl]])
            st[1][sl] = plsc.bitcast(st[0][2, sl], jnp.float32) / dv

    vsets = ((pka, wa, rowsa, sema), (pkb, wb, rowsb, semb))
    _pipe_vector_loop(wid * NCHUNK_V, NCHUNK_V, vsets, vload, vcompw,
                      feats_hbm, acc_sh)
    plsc.subcore_barrier()
    pltpu.sync_copy(acc_sh.at[pl.ds(s * NPS, NPS)],
                    acc_out.at[c, pl.ds(s * NPS, NPS)])


_sc_pass_c = functools.partial(
    pl.kernel,
    out_type=(jax.ShapeDtypeStruct((NC, N_PAD, D), jnp.float32),
              jax.ShapeDtypeStruct((NC, N_PAD), jnp.float32)),
    mesh=_mesh,
    compiler_params=_params,
    scratch_types=[
        pltpu.VMEM((N_PAD,), jnp.float32),
        pltpu.VMEM((3, CK), jnp.int32),
        pltpu.VMEM((CK,), jnp.float32),
        pltpu.VMEM((CK, D), jnp.float32),
        pltpu.VMEM((3, CK), jnp.int32),
        pltpu.VMEM((CK,), jnp.float32),
        pltpu.VMEM((CK, D), jnp.float32),
        pltpu.VMEM((CPT,), jnp.int32),
        pltpu.VMEM((CPT,), jnp.int32),
        pltpu.VMEM((CPT,), jnp.float32),
        pltpu.VMEM((CPT,), jnp.float32),
        pltpu.VMEM_SHARED((N_PAD, D), jnp.float32),
        pltpu.VMEM_SHARED((N_PAD,), jnp.float32),
        pltpu.SemaphoreType.DMA,
        pltpu.SemaphoreType.DMA,
    ],
)(_pass_c_body)


def _edge_ef(pa_v, qa_v, pg_v, qg_v, den1_v, pk_v, sl):
    sv16 = pk_v[0, sl]
    dv16 = pk_v[1, sl]
    att = _sigmoid(plsc.load_gather(pa_v, [sv16]) + plsc.load_gather(qa_v, [dv16]))
    att = jnp.maximum(att, 1e-4)
    gate = _sigmoid(plsc.load_gather(pg_v, [sv16]) + plsc.load_gather(qg_v, [dv16]))
    sv = plsc.bitcast(pk_v[2, sl], jnp.float32) / plsc.load_gather(den1_v, [sv16])
    final = gate * att + (1.0 - gate) * sv
    return jnp.exp(final)


def _pf_scalar_body(pk_hbm, p_hbm, den1_hbm, zn_hbm,
                    ef_out, den_out,
                    pa_v, qa_v, pg_v, qg_v, den1_v,
                    pka, wa, pkb, wb, den_sh, sema, semb):
    c = lax.axis_index("c")
    s = lax.axis_index("s")
    pltpu.sync_copy(zn_hbm.at[pl.ds(s * NPS, NPS)],
                    den_sh.at[pl.ds(s * NPS, NPS)])
    pltpu.sync_copy(p_hbm.at[0], pa_v)
    pltpu.sync_copy(p_hbm.at[1], qa_v)
    pltpu.sync_copy(p_hbm.at[2], pg_v)
    pltpu.sync_copy(p_hbm.at[3], qg_v)
    pltpu.sync_copy(den1_hbm, den1_v)
    plsc.subcore_barrier()

    scid0 = s * NCHUNK_S

    def sload(st, cid):
        pltpu.async_copy(pk_hbm.at[cid], st[0], st[2])

    def swait(st, cid):
        pltpu.make_async_copy(pk_hbm.at[cid], st[0], st[2]).wait()

    def sfin(st, cid):
        for g in range(CK // L):
            sl = pl.ds(g * L, L)
            st[1][sl] = _edge_ef(pa_v, qa_v, pg_v, qg_v, den1_v, st[0], sl)
        pltpu.sync_copy(st[1], den_sh.at[st[0].at[0]], add=True)
        pltpu.sync_copy(st[1], ef_out.at[c, cid])

    ssets = ((pka, wa, sema), (pkb, wb, semb))
    sload(ssets[0], scid0)

    def sdbl(ii, _):
        for p in (0, 1):
            i = 2 * ii + p
            q = 1 - p
            sload(ssets[q], scid0 + i + 1)
            swait(ssets[p], scid0 + i)
            sfin(ssets[p], scid0 + i)
        return 0

    lax.fori_loop(0, NCHUNK_S // 2 - 1, sdbl, 0)
    sload(ssets[1], scid0 + NCHUNK_S - 1)
    swait(ssets[0], scid0 + NCHUNK_S - 2)
    sfin(ssets[0], scid0 + NCHUNK_S - 2)
    swait(ssets[1], scid0 + NCHUNK_S - 1)
    sfin(ssets[1], scid0 + NCHUNK_S - 1)
    plsc.subcore_barrier()
    pltpu.sync_copy(den_sh.at[pl.ds(s * NPS, NPS)],
                    den_out.at[c, pl.ds(s * NPS, NPS)])


_sc_pf_scalar = functools.partial(
    pl.kernel,
    out_type=(jax.ShapeDtypeStruct((NC, NCH2, CK), jnp.float32),
              jax.ShapeDtypeStruct((NC, N_PAD), jnp.float32)),
    mesh=_mesh,
    compiler_params=_params,
    scratch_types=[
        pltpu.VMEM((N_PAD,), jnp.float32),
        pltpu.VMEM((N_PAD,), jnp.float32),
        pltpu.VMEM((N_PAD,), jnp.float32),
        pltpu.VMEM((N_PAD,), jnp.float32),
        pltpu.VMEM((N_PAD,), jnp.float32),
        pltpu.VMEM((3, CK), jnp.int32),
        pltpu.VMEM((CK,), jnp.float32),
        pltpu.VMEM((3, CK), jnp.int32),
        pltpu.VMEM((CK,), jnp.float32),
        pltpu.VMEM_SHARED((N_PAD,), jnp.float32),
        pltpu.SemaphoreType.DMA,
        pltpu.SemaphoreType.DMA,
    ],
)(_pf_scalar_body)


def _pf_vector_body(feats_hbm, pk_hbm, ef_hbm, den2_hbm, zrows_hbm,
                    acc_out,
                    den2_v, pka, wa, ava, rowsa, pkb, wb, avb, rowsb,
                    acc_sh, sema, semb):
    c = lax.axis_index("c")
    s = lax.axis_index("s")
    wid = c * NS + s
    pltpu.sync_copy(zrows_hbm.at[pl.ds(s * NPS, NPS)],
                    acc_sh.at[pl.ds(s * NPS, NPS)])
    pltpu.sync_copy(den2_hbm, den2_v)
    plsc.subcore_barrier()

    def vload(st, cid):
        pltpu.sync_copy(pk_hbm.at[cid], st[0])
        pltpu.sync_copy(ef_hbm.at[c, cid], st[4])

    def vcompw(st):
        for g in range(CK // L):
            sl = pl.ds(g * L, L)
            dv = plsc.load_gather(den2_v, [st[0][0, sl]])
            st[1][sl] = st[4][sl] / dv

    vsets = ((pka, wa, rowsa, sema, ava), (pkb, wb, rowsb, semb, avb))
    _pipe_vector_loop(wid * NCHUNK_V, NCHUNK_V, vsets, vload, vcompw,
                      feats_hbm, acc_sh)
    plsc.subcore_barrier()
    pltpu.sync_copy(acc_sh.at[pl.ds(s * NPS, NPS)],
                    acc_out.at[c, pl.ds(s * NPS, NPS)])


_sc_pf_vector = functools.partial(
    pl.kernel,
    out_type=jax.ShapeDtypeStruct((NC, N_PAD, D), jnp.float32),
    mesh=_mesh,
    compiler_params=_params,
    scratch_types=[
        pltpu.VMEM((N_PAD,), jnp.float32),
        pltpu.VMEM((3, CK), jnp.int32),
        pltpu.VMEM((CK,), jnp.float32),
        pltpu.VMEM((CK,), jnp.float32),
        pltpu.VMEM((CK, D), jnp.float32),
        pltpu.VMEM((3, CK), jnp.int32),
        pltpu.VMEM((CK,), jnp.float32),
        pltpu.VMEM((CK,), jnp.float32),
        pltpu.VMEM((CK, D), jnp.float32),
        pltpu.VMEM_SHARED((N_PAD, D), jnp.float32),
        pltpu.SemaphoreType.DMA,
        pltpu.SemaphoreType.DMA,
    ],
)(_pf_vector_body)


def _tanh_body(x_ref, o_ref):
    o_ref[...] = jnp.tanh(x_ref[...])


def _tc_tanh(x):
    n = x.shape[0]
    blk = 1000
    return pl.pallas_call(
        _tanh_body,
        grid=(n // blk,),
        in_specs=[pl.BlockSpec((blk, D), lambda i: (i, 0))],
        out_specs=pl.BlockSpec((blk, D), lambda i: (i, 0)),
        out_shape=jax.ShapeDtypeStruct((n, D), jnp.float32),
    )(x)


def _pack3(a, b, v):
    bits = lax.bitcast_convert_type(v, jnp.int32)
    return jnp.stack([a.reshape(NCH2, CK), b.reshape(NCH2, CK),
                      bits.reshape(NCH2, CK)], axis=1)


def kernel(features, rel_emb, adj, r_index, r_val, k0, k1, W_attn, b_attn, W_gate, b_gate):
    f32 = jnp.float32
    src, dst = adj[0], adj[1]
    pad_i = jnp.full((E_PAD2 - E,), PAD_SRC, jnp.int32)
    pad_z = jnp.zeros((E_PAD2 - E,), jnp.int32)
    src_p = jnp.concatenate([src, pad_i])
    dst_p = jnp.concatenate([dst, pad_z])
    r0_p = jnp.concatenate([r_index[0], pad_z])
    r1_p = jnp.concatenate([r_index[1], pad_z])
    rv_p = jnp.concatenate([r_val, jnp.zeros((E_PAD2 - E,), f32)])
    zeros_nd = jnp.zeros((N_PAD, D), f32)
    zeros_n = jnp.zeros((N_PAD,), f32)

    feats = _tc_tanh(features)
    gparts = _sc_phase_a(rel_emb, _pack3(r0_p, r1_p, rv_p), zeros_nd[:R_PAD])
    g = (gparts[0] + gparts[1])[:R]
    nrm = jnp.sqrt(jnp.sum(g * g, axis=1, keepdims=True))
    t = g / jnp.maximum(nrm, 1e-12)
    t_pad = jnp.concatenate([t, jnp.zeros((R_PAD - R, D), f32)], axis=0)
    att_s = t @ jnp.concatenate([k0, k1], axis=1)
    wa = W_attn[0]
    wg = W_gate[0]
    Wp = jnp.stack([wa[:D] + wa[2 * D:], wa[D:2 * D] - wa[2 * D:],
                    wg[:D] + wg[2 * D:], wg[D:2 * D] - wg[2 * D:]], axis=1)
    ba = jnp.stack([b_attn[0] * 0.5, b_attn[0] * 0.5, b_gate[0] * 0.5, b_gate[0] * 0.5])

    outc = [feats]
    outs = [feats]
    for l in range(DEPTH):
        av = jnp.concatenate([jnp.exp(att_s[:, l]),
                              jnp.ones((E - R,), f32),
                              jnp.zeros((E_PAD2 - E,), f32)])
        pk = _pack3(src_p, dst_p, av)
        accp, den1p = _sc_pass_c(feats, pk, src_p, dst_p, av, t_pad,
                                 zeros_n, zeros_nd)
        feats = _tc_tanh((accp[0] + accp[1])[:N])
        outc.append(feats)
        denom1 = den1p[0]
        P = feats @ Wp + ba[None, :]
        P_pad = jnp.concatenate([P, jnp.zeros((N_PAD - N, 4), f32)], axis=0).T
        ef2, den2p = _sc_pf_scalar(pk, P_pad, denom1, zeros_n)
        acc2p = _sc_pf_vector(feats, pk, ef2, den2p[0], zeros_nd)
        acc2 = (acc2p[0] + acc2p[1])[:N]
        s1 = jnp.where(den2p[0][:N] > 0, 1.0, 0.0)
        outs.append(_tc_tanh(feats * s1[:, None] - acc2))
    return (jnp.concatenate(outc, axis=-1), jnp.concatenate(outs, axis=-1))


# speedup vs baseline: 21.6486x; 1.0123x over previous
import functools
import jax
import jax.numpy as jnp
from jax import lax
from jax.experimental import pallas as pl
from jax.experimental.pallas import tpu as pltpu
from jax.experimental.pallas import tpu_sc as plsc

N = 10000
E = 320000
R = 1000
D = 128
DEPTH = 2

NC = 2
NS = 16
L = 16

CK = 128
EPT = 10112
E_PAD = EPT * NC * NS
E_PAD2 = E_PAD + CK
NCH2 = E_PAD2 // CK
EPSC = E_PAD // NS
NCHUNK_V = EPT // CK
NCHUNK_S = EPSC // CK
N_PAD = 10240
NPS = N_PAD // NS
R_PAD = 1024
RPS = R_PAD // NS
CPT = R_PAD // (NC * NS)
PAD_SRC = N_PAD - 1

_mesh = plsc.VectorSubcoreMesh(core_axis_name="c", subcore_axis_name="s",
                               num_cores=NC, num_subcores=NS)
_params = pltpu.CompilerParams(needs_layout_passes=False)


def _sigmoid(x):
    return 1.0 / (1.0 + jnp.exp(-x))


def _scale_rows(rows_v, w_v, nrows):
    UNR = 4

    def body(jj, _):
        j0 = jj * UNR
        for u in range(UNR):
            j = j0 + u
            wbc = plsc.load_gather(w_v, [jnp.full((L,), j, jnp.int32)])
            for k in range(D // L):
                sl = pl.ds(k * L, L)
                rows_v[j, sl] = rows_v[j, sl] * wbc
        return 0

    lax.fori_loop(0, nrows // UNR, body, 0)


def _pipe_vector_loop(cid0, nchunk, sets, load_idx, compute_w, feats_hbm,
                      acc_sh):
    def start_gather(st):
        pltpu.async_copy(feats_hbm.at[st[0].at[1]], st[2], st[3])

    def wait_gather(st):
        pltpu.make_async_copy(feats_hbm.at[st[0].at[1]], st[2], st[3]).wait()

    def finish(st):
        compute_w(st)
        wait_gather(st)
        _scale_rows(st[2], st[1], CK)
        pltpu.sync_copy(st[2], acc_sh.at[st[0].at[0]], add=True)

    load_idx(sets[0], cid0)
    start_gather(sets[0])

    def dbl(ii, _):
        for p in (0, 1):
            i = 2 * ii + p
            q = 1 - p
            load_idx(sets[q], cid0 + i + 1)
            start_gather(sets[q])
            finish(sets[p])
        return 0

    lax.fori_loop(0, (nchunk - 1) // 2, dbl, 0)
    finish(sets[(nchunk - 1) % 2])


def _phase_a_body(rel_hbm, pk_hbm, zrows_hbm, out_hbm,
                  pka, wa, rowsa, pkb, wb, rowsb, acc_sh, sema, semb):
    c = lax.axis_index("c")
    s = lax.axis_index("s")
    wid = c * NS + s
    pltpu.sync_copy(zrows_hbm.at[pl.ds(s * RPS, RPS)],
                    acc_sh.at[pl.ds(s * RPS, RPS)])
    plsc.subcore_barrier()

    def load_idx(st, cid):
        pltpu.sync_copy(pk_hbm.at[cid], st[0])

    def compw(st):
        for g in range(CK // L):
            sl = pl.ds(g * L, L)
            st[1][sl] = plsc.bitcast(st[0][2, sl], jnp.float32)

    sets = ((pka, wa, rowsa, sema), (pkb, wb, rowsb, semb))
    _pipe_vector_loop(wid * NCHUNK_V, NCHUNK_V, sets, load_idx, compw,
                      rel_hbm, acc_sh)
    plsc.subcore_barrier()
    pltpu.sync_copy(acc_sh.at[pl.ds(s * RPS, RPS)],
                    out_hbm.at[c, pl.ds(s * RPS, RPS)])


_sc_phase_a = functools.partial(
    pl.kernel,
    out_type=jax.ShapeDtypeStruct((NC, R_PAD, D), jnp.float32),
    mesh=_mesh,
    compiler_params=_params,
    scratch_types=[
        pltpu.VMEM((3, CK), jnp.int32),
        pltpu.VMEM((CK,), jnp.float32),
        pltpu.VMEM((CK, D), jnp.float32),
        pltpu.VMEM((3, CK), jnp.int32),
        pltpu.VMEM((CK,), jnp.float32),
        pltpu.VMEM((CK, D), jnp.float32),
        pltpu.VMEM_SHARED((R_PAD, D), jnp.float32),
        pltpu.SemaphoreType.DMA,
        pltpu.SemaphoreType.DMA,
    ],
)(_phase_a_body)


def _pass_c_body(feats_hbm, pk_hbm, src_hbm, dst_hbm, av_hbm, t_hbm,
                 zn_hbm, zrows_hbm, acc_out, den_out,
                 denom_v, pka, wa, rowsa, pkb, wb, rowsb,
                 csrc_v, cdst_v, cav_v, cw_v,
                 acc_sh, den_sh, sema, semb):
    c = lax.axis_index("c")
    s = lax.axis_index("s")
    wid = c * NS + s
    pltpu.sync_copy(zrows_hbm.at[pl.ds(s * NPS, NPS)],
                    acc_sh.at[pl.ds(s * NPS, NPS)])
    pltpu.sync_copy(zn_hbm.at[pl.ds(s * NPS, NPS)],
                    den_sh.at[pl.ds(s * NPS, NPS)])
    plsc.subcore_barrier()

    scid0 = s * NCHUNK_S

    def sload(st, cid):
        pltpu.async_copy(pk_hbm.at[cid], st[0], st[2])

    def swait(st, cid):
        pltpu.make_async_copy(pk_hbm.at[cid], st[0], st[2]).wait()

    def sfin(st):
        for g in range(CK // L):
            sl = pl.ds(g * L, L)
            st[1][sl] = plsc.bitcast(st[0][2, sl], jnp.float32)
        pltpu.sync_copy(st[1], den_sh.at[st[0].at[0]], add=True)

    ssets = ((pka, wa, sema), (pkb, wb, semb))
    sload(ssets[0], scid0)

    def sdbl(ii, _):
        for p in (0, 1):
            i = 2 * ii + p
            q = 1 - p
            sload(ssets[q], scid0 + i + 1)
            swait(ssets[p], scid0 + i)
            sfin(ssets[p])
        return 0

    lax.fori_loop(0, NCHUNK_S // 2 - 1, sdbl, 0)
    sload(ssets[1], scid0 + NCHUNK_S - 1)
    swait(ssets[0], scid0 + NCHUNK_S - 2)
    sfin(ssets[0])
    swait(ssets[1], scid0 + NCHUNK_S - 1)
    sfin(ssets[1])
    plsc.subcore_barrier()
    pltpu.sync_copy(den_sh, denom_v)
    pltpu.sync_copy(den_sh.at[pl.ds(s * NPS, NPS)],
                    den_out.at[c, pl.ds(s * NPS, NPS)])

    cbase = wid * CPT
    pltpu.sync_copy(src_hbm.at[pl.ds(cbase, CPT)], csrc_v)
    pltpu.sync_copy(dst_hbm.at[pl.ds(cbase, CPT)], cdst_v)
    pltpu.sync_copy(av_hbm.at[pl.ds(cbase, CPT)], cav_v)
    pltpu.sync_copy(t_hbm.at[pl.ds(cbase, CPT)], rowsb.at[pl.ds(0, CPT)])
    pltpu.async_copy(feats_hbm.at[cdst_v], rowsa.at[pl.ds(0, CPT)], sema).wait()
    for g in range(CPT // L):
        sl = pl.ds(g * L, L)
        dv = plsc.load_gather(denom_v, [csrc_v[sl]])
        cw_v[sl] = cav_v[sl] / dv

    def corr(j, _):
        dot = jnp.zeros((L,), jnp.float32)
        for k in range(D // L):
            sl = pl.ds(k * L, L)
            dot = dot + rowsa[j, sl] * rowsb[j, sl]
        dsc = jnp.sum(dot, axis=0)
        svbc = plsc.load_gather(cw_v, [jnp.full((L,), j, jnp.int32)])
        coef = -2.0 * dsc * svbc
        for k in range(D // L):
            sl = pl.ds(k * L, L)
            rowsa[j, sl] = coef * rowsb[j, sl]
        return 0

    lax.fori_loop(0, CPT, corr, 0)
    pltpu.sync_copy(rowsa.at[pl.ds(0, CPT)], acc_sh.at[csrc_v], add=True)

    def vload(st, cid):
        pltpu.sync_copy(pk_hbm.at[cid], st[0])

    def vcompw(st):
        for g in range(CK // L):
            sl = pl.ds(g * L, L)
            dv = plsc.load_gather(denom_v, [st[0][0, sl]])
            st[1][sl] = plsc.bitcast(st[0][2, sl], jnp.float32) / dv

    vsets = ((pka, wa, rowsa, sema), (pkb, wb, rowsb, semb))
    _pipe_vector_loop(wid * NCHUNK_V, NCHUNK_V, vsets, vload, vcompw,
                      feats_hbm, acc_sh)
    plsc.subcore_barrier()
    pltpu.sync_copy(acc_sh.at[pl.ds(s * NPS, NPS)],
                    acc_out.at[c, pl.ds(s * NPS, NPS)])


_sc_pass_c = functools.partial(
    pl.kernel,
    out_type=(jax.ShapeDtypeStruct((NC, N_PAD, D), jnp.float32),
              jax.ShapeDtypeStruct((NC, N_PAD), jnp.float32)),
    mesh=_mesh,
    compiler_params=_params,
    scratch_types=[
        pltpu.VMEM((N_PAD,), jnp.float32),
        pltpu.VMEM((3, CK), jnp.int32),
        pltpu.VMEM((CK,), jnp.float32),
        pltpu.VMEM((CK, D), jnp.float32),
        pltpu.VMEM((3, CK), jnp.int32),
        pltpu.VMEM((CK,), jnp.float32),
        pltpu.VMEM((CK, D), jnp.float32),
        pltpu.VMEM((CPT,), jnp.int32),
        pltpu.VMEM((CPT,), jnp.int32),
        pltpu.VMEM((CPT,), jnp.float32),
        pltpu.VMEM((CPT,), jnp.float32),
        pltpu.VMEM_SHARED((N_PAD, D), jnp.float32),
        pltpu.VMEM_SHARED((N_PAD,), jnp.float32),
        pltpu.SemaphoreType.DMA,
        pltpu.SemaphoreType.DMA,
    ],
)(_pass_c_body)


def _edge_ef(pa_v, qa_v, pg_v, qg_v, den1_v, pk_v, sl):
    sv16 = pk_v[0, sl]
    dv16 = pk_v[1, sl]
    att = _sigmoid(plsc.load_gather(pa_v, [sv16]) + plsc.load_gather(qa_v, [dv16]))
    att = jnp.maximum(att, 1e-4)
    gate = _sigmoid(plsc.load_gather(pg_v, [sv16]) + plsc.load_gather(qg_v, [dv16]))
    sv = plsc.bitcast(pk_v[2, sl], jnp.float32) / plsc.load_gather(den1_v, [sv16])
    final = gate * att + (1.0 - gate) * sv
    return jnp.exp(final)


def _pf_scalar_body(pk_hbm, p_hbm, den1_hbm, zn_hbm,
                    ef_out, den_out,
                    pa_v, qa_v, pg_v, qg_v, den1_v,
                    pka, wa, pkb, wb, den_sh, sema, semb):
    c = lax.axis_index("c")
    s = lax.axis_index("s")
    pltpu.sync_copy(zn_hbm.at[pl.ds(s * NPS, NPS)],
                    den_sh.at[pl.ds(s * NPS, NPS)])
    pltpu.sync_copy(p_hbm.at[0], pa_v)
    pltpu.sync_copy(p_hbm.at[1], qa_v)
    pltpu.sync_copy(p_hbm.at[2], pg_v)
    pltpu.sync_copy(p_hbm.at[3], qg_v)
    pltpu.sync_copy(den1_hbm, den1_v)
    plsc.subcore_barrier()

    scid0 = s * NCHUNK_S

    def sload(st, cid):
        pltpu.async_copy(pk_hbm.at[cid], st[0], st[2])

    def swait(st, cid):
        pltpu.make_async_copy(pk_hbm.at[cid], st[0], st[2]).wait()

    def sfin(st, cid):
        for g in range(CK // L):
            sl = pl.ds(g * L, L)
            st[1][sl] = _edge_ef(pa_v, qa_v, pg_v, qg_v, den1_v, st[0], sl)
        pltpu.sync_copy(st[1], den_sh.at[st[0].at[0]], add=True)
        pltpu.sync_copy(st[1], ef_out.at[c, cid])

    ssets = ((pka, wa, sema), (pkb, wb, semb))
    sload(ssets[0], scid0)

    def sdbl(ii, _):
        for p in (0, 1):
            i = 2 * ii + p
            q = 1 - p
            sload(ssets[q], scid0 + i + 1)
            swait(ssets[p], scid0 + i)
            sfin(ssets[p], scid0 + i)
        return 0

    lax.fori_loop(0, NCHUNK_S // 2 - 1, sdbl, 0)
    sload(ssets[1], scid0 + NCHUNK_S - 1)
    swait(ssets[0], scid0 + NCHUNK_S - 2)
    sfin(ssets[0], scid0 + NCHUNK_S - 2)
    swait(ssets[1], scid0 + NCHUNK_S - 1)
    sfin(ssets[1], scid0 + NCHUNK_S - 1)
    plsc.subcore_barrier()
    pltpu.sync_copy(den_sh.at[pl.ds(s * NPS, NPS)],
                    den_out.at[c, pl.ds(s * NPS, NPS)])


_sc_pf_scalar = functools.partial(
    pl.kernel,
    out_type=(jax.ShapeDtypeStruct((NC, NCH2, CK), jnp.float32),
              jax.ShapeDtypeStruct((NC, N_PAD), jnp.float32)),
    mesh=_mesh,
    compiler_params=_params,
    scratch_types=[
        pltpu.VMEM((N_PAD,), jnp.float32),
        pltpu.VMEM((N_PAD,), jnp.float32),
        pltpu.VMEM((N_PAD,), jnp.float32),
        pltpu.VMEM((N_PAD,), jnp.float32),
        pltpu.VMEM((N_PAD,), jnp.float32),
        pltpu.VMEM((3, CK), jnp.int32),
        pltpu.VMEM((CK,), jnp.float32),
        pltpu.VMEM((3, CK), jnp.int32),
        pltpu.VMEM((CK,), jnp.float32),
        pltpu.VMEM_SHARED((N_PAD,), jnp.float32),
        pltpu.SemaphoreType.DMA,
        pltpu.SemaphoreType.DMA,
    ],
)(_pf_scalar_body)


def _pf_vector_body(feats_hbm, pk_hbm, ef_hbm, den2_hbm, zrows_hbm,
                    acc_out,
                    den2_v, pka, wa, ava, rowsa, pkb, wb, avb, rowsb,
                    acc_sh, sema, semb):
    c = lax.axis_index("c")
    s = lax.axis_index("s")
    wid = c * NS + s
    pltpu.sync_copy(zrows_hbm.at[pl.ds(s * NPS, NPS)],
                    acc_sh.at[pl.ds(s * NPS, NPS)])
    pltpu.sync_copy(den2_hbm, den2_v)
    plsc.subcore_barrier()

    def vload(st, cid):
        pltpu.sync_copy(pk_hbm.at[cid], st[0])
        pltpu.sync_copy(ef_hbm.at[c, cid], st[4])

    def vcompw(st):
        for g in range(CK // L):
            sl = pl.ds(g * L, L)
            dv = plsc.load_gather(den2_v, [st[0][0, sl]])
            st[1][sl] = st[4][sl] / dv

    vsets = ((pka, wa, rowsa, sema, ava), (pkb, wb, rowsb, semb, avb))
    _pipe_vector_loop(wid * NCHUNK_V, NCHUNK_V, vsets, vload, vcompw,
                      feats_hbm, acc_sh)
    plsc.subcore_barrier()
    pltpu.sync_copy(acc_sh.at[pl.ds(s * NPS, NPS)],
                    acc_out.at[c, pl.ds(s * NPS, NPS)])


_sc_pf_vector = functools.partial(
    pl.kernel,
    out_type=jax.ShapeDtypeStruct((NC, N_PAD, D), jnp.float32),
    mesh=_mesh,
    compiler_params=_params,
    scratch_types=[
        pltpu.VMEM((N_PAD,), jnp.float32),
        pltpu.VMEM((3, CK), jnp.int32),
        pltpu.VMEM((CK,), jnp.float32),
        pltpu.VMEM((CK,), jnp.float32),
        pltpu.VMEM((CK, D), jnp.float32),
        pltpu.VMEM((3, CK), jnp.int32),
        pltpu.VMEM((CK,), jnp.float32),
        pltpu.VMEM((CK,), jnp.float32),
        pltpu.VMEM((CK, D), jnp.float32),
        pltpu.VMEM_SHARED((N_PAD, D), jnp.float32),
        pltpu.SemaphoreType.DMA,
        pltpu.SemaphoreType.DMA,
    ],
)(_pf_vector_body)


def _tanh_body(x_ref, o_ref):
    o_ref[...] = jnp.tanh(x_ref[...])


def _tc_tanh(x):
    n = x.shape[0]
    blk = 1000
    return pl.pallas_call(
        _tanh_body,
        grid=(n // blk,),
        in_specs=[pl.BlockSpec((blk, D), lambda i: (i, 0))],
        out_specs=pl.BlockSpec((blk, D), lambda i: (i, 0)),
        out_shape=jax.ShapeDtypeStruct((n, D), jnp.float32),
    )(x)


def _pack3(a, b, v):
    bits = lax.bitcast_convert_type(v, jnp.int32)
    return jnp.stack([a.reshape(NCH2, CK), b.reshape(NCH2, CK),
                      bits.reshape(NCH2, CK)], axis=1)


def kernel(features, rel_emb, adj, r_index, r_val, k0, k1, W_attn, b_attn, W_gate, b_gate):
    f32 = jnp.float32
    src, dst = adj[0], adj[1]
    pad_i = jnp.full((E_PAD2 - E,), PAD_SRC, jnp.int32)
    pad_z = jnp.zeros((E_PAD2 - E,), jnp.int32)
    src_p = jnp.concatenate([src, pad_i])
    dst_p = jnp.concatenate([dst, pad_z])
    r0_p = jnp.concatenate([r_index[0], pad_z])
    r1_p = jnp.concatenate([r_index[1], pad_z])
    rv_p = jnp.concatenate([r_val, jnp.zeros((E_PAD2 - E,), f32)])
    zeros_nd = jnp.zeros((N_PAD, D), f32)
    zeros_n = jnp.zeros((N_PAD,), f32)

    feats = _tc_tanh(features)
    gparts = _sc_phase_a(rel_emb, _pack3(r0_p, r1_p, rv_p), zeros_nd[:R_PAD])
    g = (gparts[0] + gparts[1])[:R]
    nrm = jnp.sqrt(jnp.sum(g * g, axis=1, keepdims=True))
    t = g / jnp.maximum(nrm, 1e-12)
    t_pad = jnp.concatenate([t, jnp.zeros((R_PAD - R, D), f32)], axis=0)
    att_s = t @ jnp.concatenate([k0, k1], axis=1)
    wa = W_attn[0]
    wg = W_gate[0]
    Wp = jnp.stack([wa[:D] + wa[2 * D:], wa[D:2 * D] - wa[2 * D:],
                    wg[:D] + wg[2 * D:], wg[D:2 * D] - wg[2 * D:]], axis=1)
    ba = jnp.stack([b_attn[0] * 0.5, b_attn[0] * 0.5, b_gate[0] * 0.5, b_gate[0] * 0.5])

    outc = [feats]
    outs = [feats]
    for l in range(DEPTH):
        av = jnp.concatenate([jnp.exp(att_s[:, l]),
                              jnp.ones((E - R,), f32),
                              jnp.zeros((E_PAD2 - E,), f32)])
        pk = _pack3(src_p, dst_p, av)
        accp, den1p = _sc_pass_c(feats, pk, src_p, dst_p, av, t_pad,
                                 zeros_n, zeros_nd)
        feats = _tc_tanh((accp[0] + accp[1])[:N])
        outc.append(feats)
        denom1 = den1p[0]
        P = feats @ Wp + ba[None, :]
        P_pad = jnp.concatenate([P, jnp.zeros((N_PAD - N, 4), f32)], axis=0).T
        ef2, den2p = _sc_pf_scalar(pk, P_pad, denom1, zeros_n)
        acc2p = _sc_pf_vector(feats, pk, ef2, den2p[0], zeros_nd)
        acc2 = (acc2p[0] + acc2p[1])[:N]
        s1 = jnp.where(den2p[0][:N] > 0, 1.0, 0.0)
        outs.append(_tc_tanh(feats * s1[:, None] - acc2))
    return (jnp.concatenate(outc, axis=-1), jnp.concatenate(outs, axis=-1))


# speedup vs baseline: 22.0092x; 1.0167x over previous
import functools
import jax
import jax.numpy as jnp
from jax import lax
from jax.experimental import pallas as pl
from jax.experimental.pallas import tpu as pltpu
from jax.experimental.pallas import tpu_sc as plsc

N = 10000
E = 320000
R = 1000
D = 128
DEPTH = 2

NC = 2
NS = 16
L = 16

CK = 128
EPT = 10112
E_PAD = EPT * NC * NS
E_PAD2 = E_PAD + CK
NCH2 = E_PAD2 // CK
EPSC = E_PAD // NS
NCHUNK_V = EPT // CK
NCHUNK_S = EPSC // CK
N_PAD = 10240
NPS = N_PAD // NS
R_PAD = 1024
RPS = R_PAD // NS
CPT = R_PAD // (NC * NS)
PAD_SRC = N_PAD - 1

_mesh = plsc.VectorSubcoreMesh(core_axis_name="c", subcore_axis_name="s",
                               num_cores=NC, num_subcores=NS)
_params = pltpu.CompilerParams(needs_layout_passes=False)


def _sigmoid(x):
    return 1.0 / (1.0 + jnp.exp(-x))


def _scale_rows(rows_v, w_v, nrows):
    UNR = 4

    def body(jj, _):
        j0 = jj * UNR
        for u in range(UNR):
            j = j0 + u
            wbc = plsc.load_gather(w_v, [jnp.full((L,), j, jnp.int32)])
            for k in range(D // L):
                sl = pl.ds(k * L, L)
                rows_v[j, sl] = rows_v[j, sl] * wbc
        return 0

    lax.fori_loop(0, nrows // UNR, body, 0)


def _pipe_vector_loop(cid0, nchunk, sets, load_idx, compute_w, feats_hbm,
                      acc_sh):
    def start_gather(st):
        pltpu.async_copy(feats_hbm.at[st[0].at[1]], st[2], st[3])

    def wait_gather(st):
        pltpu.make_async_copy(feats_hbm.at[st[0].at[1]], st[2], st[3]).wait()

    def finish(st):
        compute_w(st)
        wait_gather(st)
        _scale_rows(st[2], st[1], CK)
        pltpu.sync_copy(st[2], acc_sh.at[st[0].at[0]], add=True)

    load_idx(sets[0], cid0)
    start_gather(sets[0])

    def dbl(ii, _):
        for p in (0, 1):
            i = 2 * ii + p
            q = 1 - p
            load_idx(sets[q], cid0 + i + 1)
            start_gather(sets[q])
            finish(sets[p])
        return 0

    lax.fori_loop(0, (nchunk - 1) // 2, dbl, 0)
    finish(sets[(nchunk - 1) % 2])


def _phase_a_body(rel_hbm, pk_hbm, zrows_hbm, out_hbm,
                  pka, wa, rowsa, pkb, wb, rowsb, acc_sh, sema, semb):
    c = lax.axis_index("c")
    s = lax.axis_index("s")
    wid = c * NS + s
    pltpu.sync_copy(zrows_hbm.at[pl.ds(s * RPS, RPS)],
                    acc_sh.at[pl.ds(s * RPS, RPS)])
    plsc.subcore_barrier()

    def load_idx(st, cid):
        pltpu.sync_copy(pk_hbm.at[cid], st[0])

    def compw(st):
        for g in range(CK // L):
            sl = pl.ds(g * L, L)
            st[1][sl] = plsc.bitcast(st[0][2, sl], jnp.float32)

    sets = ((pka, wa, rowsa, sema), (pkb, wb, rowsb, semb))
    _pipe_vector_loop(wid * NCHUNK_V, NCHUNK_V, sets, load_idx, compw,
                      rel_hbm, acc_sh)
    plsc.subcore_barrier()
    pltpu.sync_copy(acc_sh.at[pl.ds(s * RPS, RPS)],
                    out_hbm.at[c, pl.ds(s * RPS, RPS)])


_sc_phase_a = functools.partial(
    pl.kernel,
    out_type=jax.ShapeDtypeStruct((NC, R_PAD, D), jnp.float32),
    mesh=_mesh,
    compiler_params=_params,
    scratch_types=[
        pltpu.VMEM((3, CK), jnp.int32),
        pltpu.VMEM((CK,), jnp.float32),
        pltpu.VMEM((CK, D), jnp.float32),
        pltpu.VMEM((3, CK), jnp.int32),
        pltpu.VMEM((CK,), jnp.float32),
        pltpu.VMEM((CK, D), jnp.float32),
        pltpu.VMEM_SHARED((R_PAD, D), jnp.float32),
        pltpu.SemaphoreType.DMA,
        pltpu.SemaphoreType.DMA,
    ],
)(_phase_a_body)


def _pass_c_body(full_scalar, feats_hbm, pk_hbm, src_hbm, dst_hbm, av_hbm,
                 t_hbm, den_init_hbm, zrows_hbm, pkd_hbm, acc_out, den_out,
                 denom_v, pka, wa, rowsa, pkb, wb, rowsb,
                 csrc_v, cdst_v, cav_v, cw_v, pkd_v,
                 acc_sh, den_sh, sema, semb):
    c = lax.axis_index("c")
    s = lax.axis_index("s")
    wid = c * NS + s
    pltpu.sync_copy(zrows_hbm.at[pl.ds(s * NPS, NPS)],
                    acc_sh.at[pl.ds(s * NPS, NPS)])
    pltpu.sync_copy(den_init_hbm.at[pl.ds(s * NPS, NPS)],
                    den_sh.at[pl.ds(s * NPS, NPS)])
    plsc.subcore_barrier()

    if full_scalar:
        scid0 = s * NCHUNK_S

        def sload(st, cid):
            pltpu.async_copy(pk_hbm.at[cid], st[0], st[2])

        def swait(st, cid):
            pltpu.make_async_copy(pk_hbm.at[cid], st[0], st[2]).wait()

        def sfin(st):
            for g in range(CK // L):
                sl = pl.ds(g * L, L)
                st[1][sl] = plsc.bitcast(st[0][2, sl], jnp.float32)
            pltpu.sync_copy(st[1], den_sh.at[st[0].at[0]], add=True)

        ssets = ((pka, wa, sema), (pkb, wb, semb))
        sload(ssets[0], scid0)

        def sdbl(ii, _):
            for p in (0, 1):
                i = 2 * ii + p
                q = 1 - p
                sload(ssets[q], scid0 + i + 1)
                swait(ssets[p], scid0 + i)
                sfin(ssets[p])
            return 0

        lax.fori_loop(0, NCHUNK_S // 2 - 1, sdbl, 0)
        sload(ssets[1], scid0 + NCHUNK_S - 1)
        swait(ssets[0], scid0 + NCHUNK_S - 2)
        sfin(ssets[0])
        swait(ssets[1], scid0 + NCHUNK_S - 1)
        sfin(ssets[1])
    else:
        @pl.when(s < R_PAD // CK)
        def _():
            pltpu.sync_copy(pkd_hbm.at[s], pkd_v)
            for g in range(CK // L):
                sl = pl.ds(g * L, L)
                wa[sl] = plsc.bitcast(pkd_v[1, sl], jnp.float32)
            pltpu.sync_copy(wa, den_sh.at[pkd_v.at[0]], add=True)
    plsc.subcore_barrier()
    pltpu.sync_copy(den_sh, denom_v)
    pltpu.sync_copy(den_sh.at[pl.ds(s * NPS, NPS)],
                    den_out.at[c, pl.ds(s * NPS, NPS)])

    cbase = wid * CPT
    pltpu.sync_copy(src_hbm.at[pl.ds(cbase, CPT)], csrc_v)
    pltpu.sync_copy(dst_hbm.at[pl.ds(cbase, CPT)], cdst_v)
    pltpu.sync_copy(av_hbm.at[pl.ds(cbase, CPT)], cav_v)
    pltpu.sync_copy(t_hbm.at[pl.ds(cbase, CPT)], rowsb.at[pl.ds(0, CPT)])
    pltpu.async_copy(feats_hbm.at[cdst_v], rowsa.at[pl.ds(0, CPT)], sema).wait()
    for g in range(CPT // L):
        sl = pl.ds(g * L, L)
        dv = plsc.load_gather(denom_v, [csrc_v[sl]])
        cw_v[sl] = cav_v[sl] / dv

    def corr(j, _):
        dot = jnp.zeros((L,), jnp.float32)
        for k in range(D // L):
            sl = pl.ds(k * L, L)
            dot = dot + rowsa[j, sl] * rowsb[j, sl]
        dsc = jnp.sum(dot, axis=0)
        svbc = plsc.load_gather(cw_v, [jnp.full((L,), j, jnp.int32)])
        coef = -2.0 * dsc * svbc
        for k in range(D // L):
            sl = pl.ds(k * L, L)
            rowsa[j, sl] = coef * rowsb[j, sl]
        return 0

    lax.fori_loop(0, CPT, corr, 0)
    pltpu.sync_copy(rowsa.at[pl.ds(0, CPT)], acc_sh.at[csrc_v], add=True)

    def vload(st, cid):
        pltpu.sync_copy(pk_hbm.at[cid], st[0])

    def vcompw(st):
        for g in range(CK // L):
            sl = pl.ds(g * L, L)
            dv = plsc.load_gather(denom_v, [st[0][0, sl]])
            st[1][sl] = plsc.bitcast(st[0][2, sl], jnp.float32) / dv

    vsets = ((pka, wa, rowsa, sema), (pkb, wb, rowsb, semb))
    _pipe_vector_loop(wid * NCHUNK_V, NCHUNK_V, vsets, vload, vcompw,
                      feats_hbm, acc_sh)
    plsc.subcore_barrier()
    pltpu.sync_copy(acc_sh.at[pl.ds(s * NPS, NPS)],
                    acc_out.at[c, pl.ds(s * NPS, NPS)])


def _make_pass_c(full_scalar):
    return functools.partial(
        pl.kernel,
        out_type=(jax.ShapeDtypeStruct((NC, N_PAD, D), jnp.float32),
                  jax.ShapeDtypeStruct((NC, N_PAD), jnp.float32)),
        mesh=_mesh,
        compiler_params=_params,
        scratch_types=[
            pltpu.VMEM((N_PAD,), jnp.float32),
            pltpu.VMEM((3, CK), jnp.int32),
            pltpu.VMEM((CK,), jnp.float32),
            pltpu.VMEM((CK, D), jnp.float32),
            pltpu.VMEM((3, CK), jnp.int32),
            pltpu.VMEM((CK,), jnp.float32),
            pltpu.VMEM((CK, D), jnp.float32),
            pltpu.VMEM((CPT,), jnp.int32),
            pltpu.VMEM((CPT,), jnp.int32),
            pltpu.VMEM((CPT,), jnp.float32),
            pltpu.VMEM((CPT,), jnp.float32),
            pltpu.VMEM((2, CK), jnp.int32),
            pltpu.VMEM_SHARED((N_PAD, D), jnp.float32),
            pltpu.VMEM_SHARED((N_PAD,), jnp.float32),
            pltpu.SemaphoreType.DMA,
            pltpu.SemaphoreType.DMA,
        ],
    )(functools.partial(_pass_c_body, full_scalar))


_sc_pass_c1 = _make_pass_c(True)
_sc_pass_c2 = _make_pass_c(False)


def _edge_ef(pa_v, qa_v, pg_v, qg_v, den1_v, pk_v, sl):
    sv16 = pk_v[0, sl]
    dv16 = pk_v[1, sl]
    att = _sigmoid(plsc.load_gather(pa_v, [sv16]) + plsc.load_gather(qa_v, [dv16]))
    att = jnp.maximum(att, 1e-4)
    gate = _sigmoid(plsc.load_gather(pg_v, [sv16]) + plsc.load_gather(qg_v, [dv16]))
    sv = plsc.bitcast(pk_v[2, sl], jnp.float32) / plsc.load_gather(den1_v, [sv16])
    final = gate * att + (1.0 - gate) * sv
    return jnp.exp(final)


def _pf_scalar_body(pk_hbm, p_hbm, den1_hbm, zn_hbm,
                    ef_out, den_out,
                    pa_v, qa_v, pg_v, qg_v, den1_v,
                    pka, wa, pkb, wb, den_sh, sema, semb):
    c = lax.axis_index("c")
    s = lax.axis_index("s")
    pltpu.sync_copy(zn_hbm.at[pl.ds(s * NPS, NPS)],
                    den_sh.at[pl.ds(s * NPS, NPS)])
    pltpu.sync_copy(p_hbm.at[0], pa_v)
    pltpu.sync_copy(p_hbm.at[1], qa_v)
    pltpu.sync_copy(p_hbm.at[2], pg_v)
    pltpu.sync_copy(p_hbm.at[3], qg_v)
    pltpu.sync_copy(den1_hbm, den1_v)
    plsc.subcore_barrier()

    scid0 = s * NCHUNK_S

    def sload(st, cid):
        pltpu.async_copy(pk_hbm.at[cid], st[0], st[2])

    def swait(st, cid):
        pltpu.make_async_copy(pk_hbm.at[cid], st[0], st[2]).wait()

    def sfin(st, cid):
        for g in range(CK // L):
            sl = pl.ds(g * L, L)
            st[1][sl] = _edge_ef(pa_v, qa_v, pg_v, qg_v, den1_v, st[0], sl)
        pltpu.sync_copy(st[1], den_sh.at[st[0].at[0]], add=True)
        pltpu.sync_copy(st[1], ef_out.at[c, cid])

    ssets = ((pka, wa, sema), (pkb, wb, semb))
    sload(ssets[0], scid0)

    def sdbl(ii, _):
        for p in (0, 1):
            i = 2 * ii + p
            q = 1 - p
            sload(ssets[q], scid0 + i + 1)
            swait(ssets[p], scid0 + i)
            sfin(ssets[p], scid0 + i)
        return 0

    lax.fori_loop(0, NCHUNK_S // 2 - 1, sdbl, 0)
    sload(ssets[1], scid0 + NCHUNK_S - 1)
    swait(ssets[0], scid0 + NCHUNK_S - 2)
    sfin(ssets[0], scid0 + NCHUNK_S - 2)
    swait(ssets[1], scid0 + NCHUNK_S - 1)
    sfin(ssets[1], scid0 + NCHUNK_S - 1)
    plsc.subcore_barrier()
    pltpu.sync_copy(den_sh.at[pl.ds(s * NPS, NPS)],
                    den_out.at[c, pl.ds(s * NPS, NPS)])


_sc_pf_scalar = functools.partial(
    pl.kernel,
    out_type=(jax.ShapeDtypeStruct((NC, NCH2, CK), jnp.float32),
              jax.ShapeDtypeStruct((NC, N_PAD), jnp.float32)),
    mesh=_mesh,
    compiler_params=_params,
    scratch_types=[
        pltpu.VMEM((N_PAD,), jnp.float32),
        pltpu.VMEM((N_PAD,), jnp.float32),
        pltpu.VMEM((N_PAD,), jnp.float32),
        pltpu.VMEM((N_PAD,), jnp.float32),
        pltpu.VMEM((N_PAD,), jnp.float32),
        pltpu.VMEM((3, CK), jnp.int32),
        pltpu.VMEM((CK,), jnp.float32),
        pltpu.VMEM((3, CK), jnp.int32),
        pltpu.VMEM((CK,), jnp.float32),
        pltpu.VMEM_SHARED((N_PAD,), jnp.float32),
        pltpu.SemaphoreType.DMA,
        pltpu.SemaphoreType.DMA,
    ],
)(_pf_scalar_body)


def _pf_vector_body(feats_hbm, pk_hbm, ef_hbm, den2_hbm, zrows_hbm,
                    acc_out,
                    den2_v, pka, wa, ava, rowsa, pkb, wb, avb, rowsb,
                    acc_sh, sema, semb):
    c = lax.axis_index("c")
    s = lax.axis_index("s")
    wid = c * NS + s
    pltpu.sync_copy(zrows_hbm.at[pl.ds(s * NPS, NPS)],
                    acc_sh.at[pl.ds(s * NPS, NPS)])
    pltpu.sync_copy(den2_hbm, den2_v)
    plsc.subcore_barrier()

    def vload(st, cid):
        pltpu.sync_copy(pk_hbm.at[cid], st[0])
        pltpu.sync_copy(ef_hbm.at[c, cid], st[4])

    def vcompw(st):
        for g in range(CK // L):
            sl = pl.ds(g * L, L)
            dv = plsc.load_gather(den2_v, [st[0][0, sl]])
            st[1][sl] = st[4][sl] / dv

    vsets = ((pka, wa, rowsa, sema, ava), (pkb, wb, rowsb, semb, avb))
    _pipe_vector_loop(wid * NCHUNK_V, NCHUNK_V, vsets, vload, vcompw,
                      feats_hbm, acc_sh)
    plsc.subcore_barrier()
    pltpu.sync_copy(acc_sh.at[pl.ds(s * NPS, NPS)],
                    acc_out.at[c, pl.ds(s * NPS, NPS)])


_sc_pf_vector = functools.partial(
    pl.kernel,
    out_type=jax.ShapeDtypeStruct((NC, N_PAD, D), jnp.float32),
    mesh=_mesh,
    compiler_params=_params,
    scratch_types=[
        pltpu.VMEM((N_PAD,), jnp.float32),
        pltpu.VMEM((3, CK), jnp.int32),
        pltpu.VMEM((CK,), jnp.float32),
        pltpu.VMEM((CK,), jnp.float32),
        pltpu.VMEM((CK, D), jnp.float32),
        pltpu.VMEM((3, CK), jnp.int32),
        pltpu.VMEM((CK,), jnp.float32),
        pltpu.VMEM((CK,), jnp.float32),
        pltpu.VMEM((CK, D), jnp.float32),
        pltpu.VMEM_SHARED((N_PAD, D), jnp.float32),
        pltpu.SemaphoreType.DMA,
        pltpu.SemaphoreType.DMA,
    ],
)(_pf_vector_body)


def _tanh_body(x_ref, o_ref):
    o_ref[...] = jnp.tanh(x_ref[...])


def _tc_tanh(x):
    n = x.shape[0]
    blk = 1000
    return pl.pallas_call(
        _tanh_body,
        grid=(n // blk,),
        in_specs=[pl.BlockSpec((blk, D), lambda i: (i, 0))],
        out_specs=pl.BlockSpec((blk, D), lambda i: (i, 0)),
        out_shape=jax.ShapeDtypeStruct((n, D), jnp.float32),
    )(x)


def _pack3(a, b, v):
    bits = lax.bitcast_convert_type(v, jnp.int32)
    return jnp.stack([a.reshape(NCH2, CK), b.reshape(NCH2, CK),
                      bits.reshape(NCH2, CK)], axis=1)


def kernel(features, rel_emb, adj, r_index, r_val, k0, k1, W_attn, b_attn, W_gate, b_gate):
    f32 = jnp.float32
    src, dst = adj[0], adj[1]
    pad_i = jnp.full((E_PAD2 - E,), PAD_SRC, jnp.int32)
    pad_z = jnp.zeros((E_PAD2 - E,), jnp.int32)
    src_p = jnp.concatenate([src, pad_i])
    dst_p = jnp.concatenate([dst, pad_z])
    r0_p = jnp.concatenate([r_index[0], pad_z])
    r1_p = jnp.concatenate([r_index[1], pad_z])
    rv_p = jnp.concatenate([r_val, jnp.zeros((E_PAD2 - E,), f32)])
    zeros_nd = jnp.zeros((N_PAD, D), f32)
    zeros_n = jnp.zeros((N_PAD,), f32)

    feats = _tc_tanh(features)
    gparts = _sc_phase_a(rel_emb, _pack3(r0_p, r1_p, rv_p), zeros_nd[:R_PAD])
    g = (gparts[0] + gparts[1])[:R]
    nrm = jnp.sqrt(jnp.sum(g * g, axis=1, keepdims=True))
    t = g / jnp.maximum(nrm, 1e-12)
    t_pad = jnp.concatenate([t, jnp.zeros((R_PAD - R, D), f32)], axis=0)
    att_s = t @ jnp.concatenate([k0, k1], axis=1)
    wa = W_attn[0]
    wg = W_gate[0]
    Wp = jnp.stack([wa[:D] + wa[2 * D:], wa[D:2 * D] - wa[2 * D:],
                    wg[:D] + wg[2 * D:], wg[D:2 * D] - wg[2 * D:]], axis=1)
    ba = jnp.stack([b_attn[0] * 0.5, b_attn[0] * 0.5, b_gate[0] * 0.5, b_gate[0] * 0.5])

    pkd_dummy = jnp.zeros((R_PAD // CK, 2, CK), jnp.int32)
    av_prev = None
    denom1 = zeros_n
    outc = [feats]
    outs = [feats]
    for l in range(DEPTH):
        av = jnp.concatenate([jnp.exp(att_s[:, l]),
                              jnp.ones((E - R,), f32),
                              jnp.zeros((E_PAD2 - E,), f32)])
        pk = _pack3(src_p, dst_p, av)
        if l == 0:
            accp, den1p = _sc_pass_c1(feats, pk, src_p, dst_p, av, t_pad,
                                      zeros_n, zeros_nd, pkd_dummy)
        else:
            avd = av[:R_PAD] - av_prev[:R_PAD]
            pkd = jnp.stack([src_p[:R_PAD].reshape(R_PAD // CK, CK),
                             lax.bitcast_convert_type(avd, jnp.int32)
                             .reshape(R_PAD // CK, CK)], axis=1)
            accp, den1p = _sc_pass_c2(feats, pk, src_p, dst_p, av, t_pad,
                                      denom1, zeros_nd, pkd)
        av_prev = av
        feats = _tc_tanh((accp[0] + accp[1])[:N])
        outc.append(feats)
        denom1 = den1p[0]
        P = feats @ Wp + ba[None, :]
        P_pad = jnp.concatenate([P, jnp.zeros((N_PAD - N, 4), f32)], axis=0).T
        ef2, den2p = _sc_pf_scalar(pk, P_pad, denom1, zeros_n)
        acc2p = _sc_pf_vector(feats, pk, ef2, den2p[0], zeros_nd)
        acc2 = (acc2p[0] + acc2p[1])[:N]
        s1 = jnp.where(den2p[0][:N] > 0, 1.0, 0.0)
        outs.append(_tc_tanh(feats * s1[:, None] - acc2))
    return (jnp.concatenate(outc, axis=-1), jnp.concatenate(outs, axis=-1))
